# Initial kernel scaffold; baseline (speedup 1.0000x reference)
import jax
import jax.numpy as jnp
from jax.experimental import pallas as pl


def kernel(feat, gallery_features):
    raise NotImplementedError("write your pallas kernel here")


# speedup vs baseline: 6.0622x; 6.0622x over previous
import functools

import jax
import jax.numpy as jnp
from jax import lax
from jax.experimental import pallas as pl
from jax.experimental.pallas import tpu as pltpu

K_NN = 64
_CHUNK = 2048
_QBLK = 16
_NEG = float("-inf")


def _f2key(x):
    b = lax.bitcast_convert_type(x, jnp.int32)
    mask = lax.shift_right_arithmetic(b, 31) & jnp.int32(0x7FFFFFFF)
    s = lax.bitwise_xor(b, mask)
    return lax.bitcast_convert_type(s, jnp.uint32) + jnp.uint32(0x80000000)


def _key2f(u):
    s = lax.bitcast_convert_type(u + jnp.uint32(0x80000000), jnp.int32)
    mask = lax.shift_right_arithmetic(s, 31) & jnp.int32(0x7FFFFFFF)
    b = lax.bitwise_xor(s, mask)
    return lax.bitcast_convert_type(b, jnp.float32)


def _sims_kernel(feat_ref, gal_ref, sims_ref, *, n_real):
    j = pl.program_id(0)
    q = feat_ref[...]
    qn = q * lax.rsqrt(jnp.maximum(jnp.sum(q * q, axis=1, keepdims=True), 1e-30))
    g = gal_ref[...]
    gn = g * lax.rsqrt(jnp.maximum(jnp.sum(g * g, axis=1, keepdims=True), 1e-30))
    s = lax.dot_general(qn, gn, (((1,), (1,)), ((), ())),
                        preferred_element_type=jnp.float32)
    col = j * _CHUNK + lax.broadcasted_iota(jnp.int32, s.shape, 1)
    sims_ref[...] = jnp.where(col < n_real, s, _NEG)


def _entropy_kernel(sims_ref, ent_ref):
    v = sims_ref[...]
    m = jnp.max(v, axis=1, keepdims=True)

    lo0 = jnp.broadcast_to(_f2key(jnp.float32(-1.5)), m.shape)
    hi0 = _f2key(m) + jnp.uint32(1)

    def body(_, carry):
        lo, hi = carry
        mid = lo + lax.shift_right_logical(hi - lo, jnp.uint32(1))
        t = _key2f(mid)
        cnt = jnp.sum((v >= t).astype(jnp.int32), axis=1, keepdims=True)
        ge = cnt >= K_NN
        return jnp.where(ge, mid, lo), jnp.where(ge, hi, mid)

    lo, _ = lax.fori_loop(0, 32, body, (lo0, hi0))
    v64 = _key2f(lo)

    gt = v > v64
    u = v - m
    e = jnp.where(gt, jnp.exp(u), 0.0)
    ue = jnp.where(gt, u * jnp.exp(u), 0.0)
    z_gt = jnp.sum(e, axis=1, keepdims=True)
    t_gt = jnp.sum(ue, axis=1, keepdims=True)
    cnt_gt = jnp.sum(gt.astype(jnp.float32), axis=1, keepdims=True)
    n_tie = jnp.float32(K_NN) - cnt_gt
    ut = v64 - m
    et = jnp.exp(ut)
    z = z_gt + n_tie * et
    tt = t_gt + n_tie * ut * et
    ent = jnp.log(z) - tt / z
    ent_ref[...] = jnp.broadcast_to(ent, ent_ref.shape)


def _mean_kernel(ent_ref, out_ref, *, denom):
    out_ref[...] = jnp.sum(ent_ref[...]).reshape(1, 1) * (1.0 / denom)


def kernel(feat, gallery_features):
    nq, d = feat.shape
    ng = gallery_features.shape[0]
    ng_pad = ((ng + _CHUNK - 1) // _CHUNK) * _CHUNK
    gal = jnp.pad(gallery_features, ((0, ng_pad - ng), (0, 0)))

    sims = pl.pallas_call(
        functools.partial(_sims_kernel, n_real=ng),
        grid=(ng_pad // _CHUNK,),
        in_specs=[
            pl.BlockSpec((nq, d), lambda j: (0, 0)),
            pl.BlockSpec((_CHUNK, d), lambda j: (j, 0)),
        ],
        out_specs=pl.BlockSpec((nq, _CHUNK), lambda j: (0, j)),
        out_shape=jax.ShapeDtypeStruct((nq, ng_pad), jnp.float32),
    )(feat, gal)

    ent = pl.pallas_call(
        _entropy_kernel,
        grid=(nq // _QBLK,),
        in_specs=[pl.BlockSpec((_QBLK, ng_pad), lambda i: (i, 0))],
        out_specs=pl.BlockSpec((_QBLK, 128), lambda i: (i, 0)),
        out_shape=jax.ShapeDtypeStruct((nq, 128), jnp.float32),
    )(sims)

    out = pl.pallas_call(
        functools.partial(_mean_kernel, denom=float(nq * 128)),
        in_specs=[pl.BlockSpec((nq, 128), lambda: (0, 0))],
        out_specs=pl.BlockSpec((1, 1), lambda: (0, 0)),
        out_shape=jax.ShapeDtypeStruct((1, 1), jnp.float32),
    )(ent)
    return out[0, 0]


# speedup vs baseline: 13.2827x; 2.1911x over previous
import functools

import jax
import jax.numpy as jnp
from jax import lax
from jax.experimental import pallas as pl
from jax.experimental.pallas import tpu as pltpu
from jax.experimental.pallas import tpu_sc as plsc

K_NN = 64
_CHUNK = 2048
_BUCKET = 128
_NEG = float("-inf")
_NC, _NS, _L = 2, 16, 16
_NW = _NC * _NS


def _f2key(x):
    b = lax.bitcast_convert_type(x, jnp.int32)
    mask = lax.shift_right_arithmetic(b, 31) & jnp.int32(0x7FFFFFFF)
    s = lax.bitwise_xor(b, mask)
    return lax.bitcast_convert_type(s, jnp.uint32) + jnp.uint32(0x80000000)


def _key2f(u):
    s = lax.bitcast_convert_type(u + jnp.uint32(0x80000000), jnp.int32)
    mask = lax.shift_right_arithmetic(s, 31) & jnp.int32(0x7FFFFFFF)
    b = lax.bitwise_xor(s, mask)
    return lax.bitcast_convert_type(b, jnp.float32)


def _sims_kernel(feat_ref, gal_ref, sims_ref, bmax_ref, *, n_real):
    j = pl.program_id(0)
    q = feat_ref[...]
    qn = q * lax.rsqrt(jnp.maximum(jnp.sum(q * q, axis=1, keepdims=True), 1e-30))
    g = gal_ref[...]
    gn = g * lax.rsqrt(jnp.maximum(jnp.sum(g * g, axis=1, keepdims=True), 1e-30))
    s = lax.dot_general(qn, gn, (((1,), (1,)), ((), ())),
                        preferred_element_type=jnp.float32)
    col = j * _CHUNK + lax.broadcasted_iota(jnp.int32, s.shape, 1)
    s = jnp.where(col < n_real, s, _NEG)
    sims_ref[...] = s
    nq = s.shape[0]
    bm = jnp.max(s.reshape(nq, _CHUNK // _BUCKET, _BUCKET), axis=2)
    bmax_ref[...] = bm.reshape(1, nq, _CHUNK // _BUCKET)


_KEY_LO = 1077936127


def _sc_select(sims2d, bmax, out, bmax_v, ids_v, idx_v, cand_v, surv_v,
               orow_v, sem, *, nq, nb):
    wid = lax.axis_index("s") * _NC + lax.axis_index("c")
    rows_per_w = nq // _NW
    nbv = nb // _L
    iota = lax.broadcasted_iota(jnp.int32, (_L,), 0)
    ninf = jnp.full((_L,), _NEG, jnp.float32)
    zf = jnp.zeros((_L,), jnp.float32)

    def popcnt(msk):
        return plsc.all_reduce_population_count(msk)

    def to_scalar_i(vec):
        return vec[0]

    def cnt_ge(ref, nvreg, t):
        def b(k, c):
            v = ref[pl.ds(k * _L, _L)]
            return c + popcnt(v >= t)
        return lax.fori_loop(0, nvreg, b, jnp.zeros((_L,), jnp.int32))

    def bis(ref, nvreg, lo0, hi0):
        def b(_, carry):
            lo, hi = carry
            mid = lo + lax.shift_right_logical(hi - lo, jnp.uint32(1))
            ge = cnt_ge(ref, nvreg, _key2f(mid)) >= K_NN
            return jnp.where(ge, mid, lo), jnp.where(ge, hi, mid)
        lo, _ = lax.fori_loop(0, 32, b, (lo0, hi0))
        return _key2f(lo)

    def row_body(i, _):
        r = wid * rows_per_w + i
        pltpu.sync_copy(bmax.at[r], bmax_v)

        def mx_body(k, acc):
            return jnp.maximum(acc, bmax_v[pl.ds(k * _L, _L)])
        mvec = lax.fori_loop(0, nbv, mx_body, ninf)
        m = mvec[0]
        for k in range(1, _L):
            m = jnp.maximum(m, mvec[k])
        m_v = jnp.full((_L,), 0.0, jnp.float32) + m

        lo0 = jnp.full((_L,), _KEY_LO, jnp.uint32)
        t_low = bis(bmax_v, nbv, lo0, _f2key(m_v) + jnp.uint32(1))

        def gt_body(k, off):
            v = bmax_v[pl.ds(k * _L, _L)]
            msk = v > t_low
            plsc.store_compressed(ids_v.at[pl.ds(off, _L)], k * _L + iota,
                                  mask=msk)
            return off + to_scalar_i(popcnt(msk))

        off = lax.fori_loop(0, nbv, gt_body, jnp.int32(0))

        def eq_body(k, off):
            v = bmax_v[pl.ds(k * _L, _L)]
            msk = v == t_low
            @pl.when(off < K_NN)
            def _():
                plsc.store_compressed(ids_v.at[pl.ds(off, _L)],
                                      k * _L + iota, mask=msk)
            return off + to_scalar_i(popcnt(msk))

        lax.fori_loop(0, nbv, eq_body, off)

        def idx_body(k, _):
            idx_v[pl.ds(k * _L, _L)] = ids_v[pl.ds(k * _L, _L)] + r * nb
            return 0
        lax.fori_loop(0, K_NN // _L, idx_body, 0)
        pltpu.async_copy(sims2d.at[idx_v], cand_v, sem).wait()

        def f_outer(j2, off):
            def f_inner(l, off):
                v = cand_v[j2, pl.ds(l * _L, _L)]
                msk = v >= t_low
                plsc.store_compressed(surv_v.at[pl.ds(off, _L)], v, mask=msk)
                return off + to_scalar_i(popcnt(msk))
            return lax.fori_loop(0, _BUCKET // _L, f_inner, off)

        n = lax.fori_loop(0, K_NN, f_outer, jnp.int32(0))
        surv_v[pl.ds(n, _L)] = ninf
        nv = lax.div(n + (_L - 1), jnp.int32(_L))

        v64 = bis(surv_v, nv, _f2key(t_low), _f2key(m_v) + jnp.uint32(1))

        def s_body(k, carry):
            zv, tv, cv = carry
            v = surv_v[pl.ds(k * _L, _L)]
            gt = v > v64
            e = jnp.exp(v - m_v)
            zv = zv + jnp.where(gt, e, 0.0)
            tv = tv + jnp.where(gt, (v - m_v) * e, 0.0)
            cv = cv + popcnt(gt)
            return zv, tv, cv

        zv, tv, cv = lax.fori_loop(0, nv, s_body,
                                   (zf, zf, jnp.zeros((_L,), jnp.int32)))
        z_gt = zv[0]
        t_gt = tv[0]
        for k in range(1, _L):
            z_gt = z_gt + zv[k]
            t_gt = t_gt + tv[k]

        orow_v[...] = ((iota == 0).astype(jnp.float32) * z_gt
                       + (iota == 1).astype(jnp.float32) * t_gt
                       + (iota == 2).astype(jnp.float32) * cv.astype(jnp.float32)
                       + (iota == 3).astype(jnp.float32) * v64
                       + (iota == 4).astype(jnp.float32) * m)
        pltpu.sync_copy(orow_v, out.at[r])
        return 0

    lax.fori_loop(0, rows_per_w, row_body, 0)


def _finish_kernel(st_ref, out_ref, *, nq):
    x = st_ref[...]
    z_gt, t_gt = x[:, 0:1], x[:, 1:2]
    c_gt, v64, m = x[:, 2:3], x[:, 3:4], x[:, 4:5]
    n_tie = jnp.float32(K_NN) - c_gt
    ut = v64 - m
    et = jnp.exp(ut)
    z = z_gt + n_tie * et
    t = t_gt + n_tie * ut * et
    ent = jnp.log(z) - t / z
    out_ref[...] = jnp.sum(ent).reshape(1, 1) * (1.0 / nq)


def kernel(feat, gallery_features):
    nq, d = feat.shape
    ng = gallery_features.shape[0]
    ng_pad = ((ng + _CHUNK - 1) // _CHUNK) * _CHUNK
    nb = ng_pad // _BUCKET
    nchunks = ng_pad // _CHUNK
    bpc = _CHUNK // _BUCKET
    gal = jnp.pad(gallery_features, ((0, ng_pad - ng), (0, 0)))

    sims, bmax3 = pl.pallas_call(
        functools.partial(_sims_kernel, n_real=ng),
        grid=(nchunks,),
        in_specs=[
            pl.BlockSpec((nq, d), lambda j: (0, 0)),
            pl.BlockSpec((_CHUNK, d), lambda j: (j, 0)),
        ],
        out_specs=[
            pl.BlockSpec((nq, _CHUNK), lambda j: (0, j)),
            pl.BlockSpec((1, nq, bpc), lambda j: (j, 0, 0)),
        ],
        out_shape=[
            jax.ShapeDtypeStruct((nq, ng_pad), jnp.float32),
            jax.ShapeDtypeStruct((nchunks, nq, bpc), jnp.float32),
        ],
    )(feat, gal)

    bmax = bmax3.transpose(1, 0, 2).reshape(nq, nb)
    sims2d = sims.reshape(nq * nb, _BUCKET)

    sc_fn = functools.partial(
        pl.kernel,
        mesh=plsc.VectorSubcoreMesh(core_axis_name="c", subcore_axis_name="s"),
        compiler_params=pltpu.CompilerParams(needs_layout_passes=False),
        out_type=jax.ShapeDtypeStruct((nq, _L), jnp.float32),
        scratch_types=[
            pltpu.VMEM((nb,), jnp.float32),
            pltpu.VMEM((K_NN + 2 * _L,), jnp.int32),
            pltpu.VMEM((K_NN,), jnp.int32),
            pltpu.VMEM((K_NN, _BUCKET), jnp.float32),
            pltpu.VMEM((K_NN * _BUCKET + _L,), jnp.float32),
            pltpu.VMEM((_L,), jnp.float32),
            pltpu.SemaphoreType.DMA,
        ],
    )(functools.partial(_sc_select, nq=nq, nb=nb))
    stats = sc_fn(sims2d, bmax)

    out = pl.pallas_call(
        functools.partial(_finish_kernel, nq=float(nq)),
        in_specs=[pl.BlockSpec((nq, _L), lambda: (0, 0))],
        out_specs=pl.BlockSpec((1, 1), lambda: (0, 0)),
        out_shape=jax.ShapeDtypeStruct((1, 1), jnp.float32),
    )(stats)
    return out[0, 0]


# speedup vs baseline: 14.2026x; 1.0693x over previous
import functools

import jax
import jax.numpy as jnp
from jax import lax
from jax.experimental import pallas as pl
from jax.experimental.pallas import tpu as pltpu
from jax.experimental.pallas import tpu_sc as plsc

K_NN = 64
_CHUNK = 2048
_BUCKET = 128
_NEG = float("-inf")
_NC, _NS, _L = 2, 16, 16
_NW = _NC * _NS


def _f2key(x):
    b = lax.bitcast_convert_type(x, jnp.int32)
    mask = lax.shift_right_arithmetic(b, 31) & jnp.int32(0x7FFFFFFF)
    s = lax.bitwise_xor(b, mask)
    return lax.bitcast_convert_type(s, jnp.uint32) + jnp.uint32(0x80000000)


def _key2f(u):
    s = lax.bitcast_convert_type(u + jnp.uint32(0x80000000), jnp.int32)
    mask = lax.shift_right_arithmetic(s, 31) & jnp.int32(0x7FFFFFFF)
    b = lax.bitwise_xor(s, mask)
    return lax.bitcast_convert_type(b, jnp.float32)


def _sims_kernel(feat_ref, gal_ref, sims_ref, bmax_ref, *, n_real):
    j = pl.program_id(0)
    q = feat_ref[...]
    qn = q * lax.rsqrt(jnp.maximum(jnp.sum(q * q, axis=1, keepdims=True), 1e-30))
    g = gal_ref[...]
    gn = g * lax.rsqrt(jnp.maximum(jnp.sum(g * g, axis=1, keepdims=True), 1e-30))
    s = lax.dot_general(qn, gn, (((1,), (1,)), ((), ())),
                        preferred_element_type=jnp.float32)
    col = j * _CHUNK + lax.broadcasted_iota(jnp.int32, s.shape, 1)
    s = jnp.where(col < n_real, s, _NEG)
    sims_ref[...] = s
    nq = s.shape[0]
    bm = jnp.max(s.reshape(nq, _CHUNK // _BUCKET, _BUCKET), axis=2)
    bmax_ref[...] = bm.reshape(1, nq, _CHUNK // _BUCKET)


_KEY_LO = 1077936127


def _sc_select(sims2d, bmax, out, bmax_v, ids_v, idx_v, cand_v, surv_v,
               orow_v, sem, *, nq, nb):
    wid = lax.axis_index("s") * _NC + lax.axis_index("c")
    rows_per_w = nq // _NW
    nbv = nb // _L
    iota = lax.broadcasted_iota(jnp.int32, (_L,), 0)
    ninf = jnp.full((_L,), _NEG, jnp.float32)
    zf = jnp.zeros((_L,), jnp.float32)

    def popcnt(msk):
        return plsc.all_reduce_population_count(msk)

    def to_scalar_i(vec):
        return vec[0]

    def cnt_ge(ref, nvreg, t):
        def b(k, c):
            v = ref[pl.ds(k * _L, _L)]
            return c + popcnt(v >= t)
        return lax.fori_loop(0, nvreg, b, jnp.zeros((_L,), jnp.int32))

    def bis_while(cnt_fn, lo0, hi0):
        def cond(carry):
            lo, hi, it = carry
            return jnp.logical_and(it < 32, (hi - lo)[0] > 1)
        def body(carry):
            lo, hi, it = carry
            mid = lo + lax.shift_right_logical(hi - lo, jnp.uint32(1))
            ge = cnt_fn(_key2f(mid)) >= K_NN
            return jnp.where(ge, mid, lo), jnp.where(ge, hi, mid), it + 1
        lo, _, _ = lax.while_loop(cond, body, (lo0, hi0, jnp.int32(0)))
        return _key2f(lo)

    def row_body(i, _):
        r = wid * rows_per_w + i
        pltpu.sync_copy(bmax.at[r], bmax_v)

        def mx_body(k, carry):
            mx, mn = carry
            v = bmax_v[pl.ds(k * _L, _L)]
            return jnp.maximum(mx, v), jnp.minimum(mn, v)
        mvec, nvec = lax.fori_loop(0, nbv - 1, mx_body,
                                   (ninf, jnp.full((_L,), jnp.inf, jnp.float32)))
        mvec = jnp.maximum(mvec, bmax_v[pl.ds((nbv - 1) * _L, _L)])
        m = mvec[0]
        lo_f = nvec[0]
        for k in range(1, _L):
            m = jnp.maximum(m, mvec[k])
            lo_f = jnp.minimum(lo_f, nvec[k])
        m_v = jnp.full((_L,), 0.0, jnp.float32) + m
        lo_v = jnp.full((_L,), 0.0, jnp.float32) + lo_f

        t_low = bis_while(lambda t: cnt_ge(bmax_v, nbv, t),
                          _f2key(lo_v), _f2key(m_v) + jnp.uint32(1))

        def gt_body(k, off):
            v = bmax_v[pl.ds(k * _L, _L)]
            msk = v > t_low
            plsc.store_compressed(ids_v.at[pl.ds(off, _L)], k * _L + iota,
                                  mask=msk)
            return off + to_scalar_i(popcnt(msk))

        off = lax.fori_loop(0, nbv, gt_body, jnp.int32(0))

        def eq_body(k, off):
            v = bmax_v[pl.ds(k * _L, _L)]
            msk = v == t_low
            @pl.when(off < K_NN)
            def _():
                plsc.store_compressed(ids_v.at[pl.ds(off, _L)],
                                      k * _L + iota, mask=msk)
            return off + to_scalar_i(popcnt(msk))

        lax.fori_loop(0, nbv, eq_body, off)

        def idx_body(k, _):
            idx_v[pl.ds(k * _L, _L)] = ids_v[pl.ds(k * _L, _L)] + r * nb
            return 0
        lax.fori_loop(0, K_NN // _L, idx_body, 0)
        pltpu.async_copy(sims2d.at[idx_v], cand_v, sem).wait()

        def f_body(k, offs):
            j2 = k >> 3
            lb = (k & 7) * _L
            new = []
            for g in range(4):
                v = cand_v[g * (K_NN // 4) + j2, pl.ds(lb, _L)]
                msk = v >= t_low
                plsc.store_compressed(surv_v.at[g, pl.ds(offs[g], _L)], v,
                                      mask=msk)
                new.append(offs[g] + to_scalar_i(popcnt(msk)))
            return tuple(new)

        z4 = (jnp.int32(0),) * 4
        offs = lax.fori_loop(0, (K_NN // 4) * (_BUCKET // _L), f_body, z4)
        for g in range(4):
            surv_v[g, pl.ds(offs[g], _L)] = ninf
        nvs = [lax.div(offs[g] + (_L - 1), jnp.int32(_L)) for g in range(4)]

        def cnt4(t):
            c = jnp.zeros((_L,), jnp.int32)
            for g in range(4):
                def b(k, c2):
                    return c2 + popcnt(surv_v[g, pl.ds(k * _L, _L)] >= t)
                c = lax.fori_loop(0, nvs[g], b, c)
            return c

        v64 = bis_while(cnt4, _f2key(t_low), _f2key(m_v) + jnp.uint32(1))

        def s_body_g(g):
            def s_body(k, carry):
                zv, tv, cv = carry
                v = surv_v[g, pl.ds(k * _L, _L)]
                gt = v > v64
                e = jnp.exp(v - m_v)
                zv = zv + jnp.where(gt, e, 0.0)
                tv = tv + jnp.where(gt, (v - m_v) * e, 0.0)
                cv = cv + popcnt(gt)
                return zv, tv, cv
            return s_body

        carry = (zf, zf, jnp.zeros((_L,), jnp.int32))
        for g in range(4):
            carry = lax.fori_loop(0, nvs[g], s_body_g(g), carry)
        zv, tv, cv = carry
        z_gt = zv[0]
        t_gt = tv[0]
        for k in range(1, _L):
            z_gt = z_gt + zv[k]
            t_gt = t_gt + tv[k]

        orow_v[...] = ((iota == 0).astype(jnp.float32) * z_gt
                       + (iota == 1).astype(jnp.float32) * t_gt
                       + (iota == 2).astype(jnp.float32) * cv.astype(jnp.float32)
                       + (iota == 3).astype(jnp.float32) * v64
                       + (iota == 4).astype(jnp.float32) * m)
        pltpu.sync_copy(orow_v, out.at[r])
        return 0

    lax.fori_loop(0, rows_per_w, row_body, 0)


def _finish_kernel(st_ref, out_ref, *, nq):
    x = st_ref[...]
    z_gt, t_gt = x[:, 0:1], x[:, 1:2]
    c_gt, v64, m = x[:, 2:3], x[:, 3:4], x[:, 4:5]
    n_tie = jnp.float32(K_NN) - c_gt
    ut = v64 - m
    et = jnp.exp(ut)
    z = z_gt + n_tie * et
    t = t_gt + n_tie * ut * et
    ent = jnp.log(z) - t / z
    out_ref[...] = jnp.sum(ent).reshape(1, 1) * (1.0 / nq)


def kernel(feat, gallery_features):
    nq, d = feat.shape
    ng = gallery_features.shape[0]
    ng_pad = ((ng + _CHUNK - 1) // _CHUNK) * _CHUNK
    nb = ng_pad // _BUCKET
    nchunks = ng_pad // _CHUNK
    bpc = _CHUNK // _BUCKET
    gal = jnp.pad(gallery_features, ((0, ng_pad - ng), (0, 0)))

    sims, bmax3 = pl.pallas_call(
        functools.partial(_sims_kernel, n_real=ng),
        grid=(nchunks,),
        in_specs=[
            pl.BlockSpec((nq, d), lambda j: (0, 0)),
            pl.BlockSpec((_CHUNK, d), lambda j: (j, 0)),
        ],
        out_specs=[
            pl.BlockSpec((nq, _CHUNK), lambda j: (0, j)),
            pl.BlockSpec((1, nq, bpc), lambda j: (j, 0, 0)),
        ],
        out_shape=[
            jax.ShapeDtypeStruct((nq, ng_pad), jnp.float32),
            jax.ShapeDtypeStruct((nchunks, nq, bpc), jnp.float32),
        ],
    )(feat, gal)

    bmax = bmax3.transpose(1, 0, 2).reshape(nq, nb)
    sims2d = sims.reshape(nq * nb, _BUCKET)

    sc_fn = functools.partial(
        pl.kernel,
        mesh=plsc.VectorSubcoreMesh(core_axis_name="c", subcore_axis_name="s"),
        compiler_params=pltpu.CompilerParams(needs_layout_passes=False),
        out_type=jax.ShapeDtypeStruct((nq, _L), jnp.float32),
        scratch_types=[
            pltpu.VMEM((nb,), jnp.float32),
            pltpu.VMEM((K_NN + 2 * _L,), jnp.int32),
            pltpu.VMEM((K_NN,), jnp.int32),
            pltpu.VMEM((K_NN, _BUCKET), jnp.float32),
            pltpu.VMEM((4, K_NN * _BUCKET // 4 + _L), jnp.float32),
            pltpu.VMEM((_L,), jnp.float32),
            pltpu.SemaphoreType.DMA,
        ],
    )(functools.partial(_sc_select, nq=nq, nb=nb))
    stats = sc_fn(sims2d, bmax)

    out = pl.pallas_call(
        functools.partial(_finish_kernel, nq=float(nq)),
        in_specs=[pl.BlockSpec((nq, _L), lambda: (0, 0))],
        out_specs=pl.BlockSpec((1, 1), lambda: (0, 0)),
        out_shape=jax.ShapeDtypeStruct((1, 1), jnp.float32),
    )(stats)
    return out[0, 0]


# speedup vs baseline: 16.8515x; 1.1865x over previous
import functools

import jax
import jax.numpy as jnp
from jax import lax
from jax.experimental import pallas as pl
from jax.experimental.pallas import tpu as pltpu
from jax.experimental.pallas import tpu_sc as plsc

K_NN = 64
_CHUNK = 2048
_BUCKET = 128
_NEG = float("-inf")
_NC, _NS, _L = 2, 16, 16
_NW = _NC * _NS


def _f2key(x):
    b = lax.bitcast_convert_type(x, jnp.int32)
    mask = lax.shift_right_arithmetic(b, 31) & jnp.int32(0x7FFFFFFF)
    s = lax.bitwise_xor(b, mask)
    return lax.bitcast_convert_type(s, jnp.uint32) + jnp.uint32(0x80000000)


def _key2f(u):
    s = lax.bitcast_convert_type(u + jnp.uint32(0x80000000), jnp.int32)
    mask = lax.shift_right_arithmetic(s, 31) & jnp.int32(0x7FFFFFFF)
    b = lax.bitwise_xor(s, mask)
    return lax.bitcast_convert_type(b, jnp.float32)


def _sims_kernel(feat_ref, gal_ref, sims_ref, bmax_ref, *, n_real):
    j = pl.program_id(0)
    q = feat_ref[...]
    qn = q * lax.rsqrt(jnp.maximum(jnp.sum(q * q, axis=1, keepdims=True), 1e-30))
    g = gal_ref[...]
    gn = g * lax.rsqrt(jnp.maximum(jnp.sum(g * g, axis=1, keepdims=True), 1e-30))
    s = lax.dot_general(qn, gn, (((1,), (1,)), ((), ())),
                        preferred_element_type=jnp.float32)
    col = j * _CHUNK + lax.broadcasted_iota(jnp.int32, s.shape, 1)
    s = jnp.where(col < n_real, s, _NEG)
    sims_ref[...] = s
    nq = s.shape[0]
    bm = jnp.max(s.reshape(nq, _CHUNK // _BUCKET, _BUCKET), axis=2)
    bmax_ref[...] = bm.reshape(1, nq, _CHUNK // _BUCKET)


_KEY_LO = 1077936127


def _sc_select(sims2d, bmax, out, bmax_v, ids_v, idx_v, cand_v, surv_v,
               orow_v, sem, *, nq, nb):
    wid = lax.axis_index("s") * _NC + lax.axis_index("c")
    rows_per_w = nq // _NW
    nbv = nb // _L
    iota = lax.broadcasted_iota(jnp.int32, (_L,), 0)
    ninf = jnp.full((_L,), _NEG, jnp.float32)
    zf = jnp.zeros((_L,), jnp.float32)

    def popcnt(msk):
        return plsc.all_reduce_population_count(msk)

    def to_scalar_i(vec):
        return vec[0]

    def cnt_ge(ref, nvreg, t):
        def b(k, c):
            v = ref[pl.ds(k * _L, _L)]
            return c + popcnt(v >= t)
        return lax.fori_loop(0, nvreg, b, jnp.zeros((_L,), jnp.int32))

    def cnt_ge_static(ref, nvreg, t):
        cs = [jnp.zeros((_L,), jnp.int32) for _ in range(4)]
        for k in range(nvreg):
            cs[k % 4] = cs[k % 4] + popcnt(ref[pl.ds(k * _L, _L)] >= t)
        return (cs[0] + cs[1]) + (cs[2] + cs[3])

    def bis_while(cnt_fn, lo0, hi0):
        def cond(carry):
            lo, hi, it = carry
            return jnp.logical_and(it < 32, (hi - lo)[0] > 1)
        def body(carry):
            lo, hi, it = carry
            mid = lo + lax.shift_right_logical(hi - lo, jnp.uint32(1))
            ge = cnt_fn(_key2f(mid)) >= K_NN
            return jnp.where(ge, mid, lo), jnp.where(ge, hi, mid), it + 1
        lo, _, _ = lax.while_loop(cond, body, (lo0, hi0, jnp.int32(0)))
        return _key2f(lo)

    def row_body(i, _):
        r = wid * rows_per_w + i
        pltpu.sync_copy(bmax.at[r], bmax_v)

        mxs = [ninf, ninf]
        mns = [jnp.full((_L,), jnp.inf, jnp.float32) for _ in range(2)]
        for k in range(nbv - 1):
            v = bmax_v[pl.ds(k * _L, _L)]
            mxs[k % 2] = jnp.maximum(mxs[k % 2], v)
            mns[k % 2] = jnp.minimum(mns[k % 2], v)
        mvec = jnp.maximum(jnp.maximum(mxs[0], mxs[1]),
                           bmax_v[pl.ds((nbv - 1) * _L, _L)])
        nvec = jnp.minimum(mns[0], mns[1])
        m = mvec[0]
        lo_f = nvec[0]
        for k in range(1, _L):
            m = jnp.maximum(m, mvec[k])
            lo_f = jnp.minimum(lo_f, nvec[k])
        m_v = jnp.full((_L,), 0.0, jnp.float32) + m
        lo_v = jnp.full((_L,), 0.0, jnp.float32) + lo_f

        t_low = bis_while(lambda t: cnt_ge_static(bmax_v, nbv, t),
                          _f2key(lo_v), _f2key(m_v) + jnp.uint32(1))

        def gt_body(k, off):
            v = bmax_v[pl.ds(k * _L, _L)]
            msk = v > t_low
            plsc.store_compressed(ids_v.at[pl.ds(off, _L)], k * _L + iota,
                                  mask=msk)
            return off + to_scalar_i(popcnt(msk))

        off = lax.fori_loop(0, nbv, gt_body, jnp.int32(0), unroll=4)

        def eq_body(k, off):
            v = bmax_v[pl.ds(k * _L, _L)]
            msk = v == t_low
            @pl.when(off < K_NN)
            def _():
                plsc.store_compressed(ids_v.at[pl.ds(off, _L)],
                                      k * _L + iota, mask=msk)
            return off + to_scalar_i(popcnt(msk))

        lax.fori_loop(0, nbv, eq_body, off, unroll=4)

        for k in range(K_NN // _L):
            idx_v[pl.ds(k * _L, _L)] = ids_v[pl.ds(k * _L, _L)] + r * nb
        pltpu.async_copy(sims2d.at[idx_v], cand_v, sem).wait()

        def f_body(k, offs):
            j2 = k >> 3
            lb = (k & 7) * _L
            new = []
            for g in range(4):
                v = cand_v[g * (K_NN // 4) + j2, pl.ds(lb, _L)]
                msk = v >= t_low
                plsc.store_compressed(surv_v.at[g, pl.ds(offs[g], _L)], v,
                                      mask=msk)
                new.append(offs[g] + to_scalar_i(popcnt(msk)))
            return tuple(new)

        z4 = (jnp.int32(0),) * 4
        offs = lax.fori_loop(0, (K_NN // 4) * (_BUCKET // _L), f_body, z4,
                             unroll=4)
        for g in range(4):
            surv_v[g, pl.ds(offs[g], _L)] = ninf
        nvs = [lax.div(offs[g] + (_L - 1), jnp.int32(_L)) for g in range(4)]

        def cnt4(t):
            c = jnp.zeros((_L,), jnp.int32)
            for g in range(4):
                def b(k, c2):
                    return c2 + popcnt(surv_v[g, pl.ds(k * _L, _L)] >= t)
                c = lax.fori_loop(0, nvs[g], b, c)
            return c

        v64 = bis_while(cnt4, _f2key(t_low), _f2key(m_v) + jnp.uint32(1))

        def s_body_g(g):
            def s_body(k, carry):
                zv, tv, cv = carry
                v = surv_v[g, pl.ds(k * _L, _L)]
                gt = v > v64
                e = jnp.exp(v - m_v)
                zv = zv + jnp.where(gt, e, 0.0)
                tv = tv + jnp.where(gt, (v - m_v) * e, 0.0)
                cv = cv + popcnt(gt)
                return zv, tv, cv
            return s_body

        carry = (zf, zf, jnp.zeros((_L,), jnp.int32))
        for g in range(4):
            carry = lax.fori_loop(0, nvs[g], s_body_g(g), carry)
        zv, tv, cv = carry
        z_gt = zv[0]
        t_gt = tv[0]
        for k in range(1, _L):
            z_gt = z_gt + zv[k]
            t_gt = t_gt + tv[k]

        orow_v[...] = ((iota == 0).astype(jnp.float32) * z_gt
                       + (iota == 1).astype(jnp.float32) * t_gt
                       + (iota == 2).astype(jnp.float32) * cv.astype(jnp.float32)
                       + (iota == 3).astype(jnp.float32) * v64
                       + (iota == 4).astype(jnp.float32) * m)
        pltpu.sync_copy(orow_v, out.at[r])
        return 0

    lax.fori_loop(0, rows_per_w, row_body, 0)


def _finish_kernel(st_ref, out_ref, *, nq):
    x = st_ref[...]
    z_gt, t_gt = x[:, 0:1], x[:, 1:2]
    c_gt, v64, m = x[:, 2:3], x[:, 3:4], x[:, 4:5]
    n_tie = jnp.float32(K_NN) - c_gt
    ut = v64 - m
    et = jnp.exp(ut)
    z = z_gt + n_tie * et
    t = t_gt + n_tie * ut * et
    ent = jnp.log(z) - t / z
    out_ref[...] = jnp.sum(ent).reshape(1, 1) * (1.0 / nq)


def kernel(feat, gallery_features):
    nq, d = feat.shape
    ng = gallery_features.shape[0]
    ng_pad = ((ng + _CHUNK - 1) // _CHUNK) * _CHUNK
    nb = ng_pad // _BUCKET
    nchunks = ng_pad // _CHUNK
    bpc = _CHUNK // _BUCKET
    gal = jnp.pad(gallery_features, ((0, ng_pad - ng), (0, 0)))

    sims, bmax3 = pl.pallas_call(
        functools.partial(_sims_kernel, n_real=ng),
        grid=(nchunks,),
        in_specs=[
            pl.BlockSpec((nq, d), lambda j: (0, 0)),
            pl.BlockSpec((_CHUNK, d), lambda j: (j, 0)),
        ],
        out_specs=[
            pl.BlockSpec((nq, _CHUNK), lambda j: (0, j)),
            pl.BlockSpec((1, nq, bpc), lambda j: (j, 0, 0)),
        ],
        out_shape=[
            jax.ShapeDtypeStruct((nq, ng_pad), jnp.float32),
            jax.ShapeDtypeStruct((nchunks, nq, bpc), jnp.float32),
        ],
    )(feat, gal)

    bmax = bmax3.transpose(1, 0, 2).reshape(nq, nb)
    sims2d = sims.reshape(nq * nb, _BUCKET)

    sc_fn = functools.partial(
        pl.kernel,
        mesh=plsc.VectorSubcoreMesh(core_axis_name="c", subcore_axis_name="s"),
        compiler_params=pltpu.CompilerParams(needs_layout_passes=False),
        out_type=jax.ShapeDtypeStruct((nq, _L), jnp.float32),
        scratch_types=[
            pltpu.VMEM((nb,), jnp.float32),
            pltpu.VMEM((K_NN + 2 * _L,), jnp.int32),
            pltpu.VMEM((K_NN,), jnp.int32),
            pltpu.VMEM((K_NN, _BUCKET), jnp.float32),
            pltpu.VMEM((4, K_NN * _BUCKET // 4 + _L), jnp.float32),
            pltpu.VMEM((_L,), jnp.float32),
            pltpu.SemaphoreType.DMA,
        ],
    )(functools.partial(_sc_select, nq=nq, nb=nb))
    stats = sc_fn(sims2d, bmax)

    out = pl.pallas_call(
        functools.partial(_finish_kernel, nq=float(nq)),
        in_specs=[pl.BlockSpec((nq, _L), lambda: (0, 0))],
        out_specs=pl.BlockSpec((1, 1), lambda: (0, 0)),
        out_shape=jax.ShapeDtypeStruct((1, 1), jnp.float32),
    )(stats)
    return out[0, 0]


# speedup vs baseline: 17.6892x; 1.0497x over previous
import functools

import jax
import jax.numpy as jnp
from jax import lax
from jax.experimental import pallas as pl
from jax.experimental.pallas import tpu as pltpu
from jax.experimental.pallas import tpu_sc as plsc

K_NN = 64
_CHUNK = 2048
_BUCKET = 128
_NEG = float("-inf")
_NC, _NS, _L = 2, 16, 16
_NW = _NC * _NS


def _f2key(x):
    b = lax.bitcast_convert_type(x, jnp.int32)
    mask = lax.shift_right_arithmetic(b, 31) & jnp.int32(0x7FFFFFFF)
    s = lax.bitwise_xor(b, mask)
    return lax.bitcast_convert_type(s, jnp.uint32) + jnp.uint32(0x80000000)


def _key2f(u):
    s = lax.bitcast_convert_type(u + jnp.uint32(0x80000000), jnp.int32)
    mask = lax.shift_right_arithmetic(s, 31) & jnp.int32(0x7FFFFFFF)
    b = lax.bitwise_xor(s, mask)
    return lax.bitcast_convert_type(b, jnp.float32)


def _sims_kernel(feat_ref, gal_ref, sims_ref, bmax_ref, *, n_real):
    j = pl.program_id(0)
    q = feat_ref[...]
    qn = q * lax.rsqrt(jnp.maximum(jnp.sum(q * q, axis=1, keepdims=True), 1e-30))
    g = gal_ref[...]
    gn = g * lax.rsqrt(jnp.maximum(jnp.sum(g * g, axis=1, keepdims=True), 1e-30))
    s = lax.dot_general(qn, gn, (((1,), (1,)), ((), ())),
                        preferred_element_type=jnp.float32)
    col = j * _CHUNK + lax.broadcasted_iota(jnp.int32, s.shape, 1)
    s = jnp.where(col < n_real, s, _NEG)
    sims_ref[...] = s
    nq = s.shape[0]
    bm = jnp.max(s.reshape(nq, _CHUNK // _BUCKET, _BUCKET), axis=2)
    bmax_ref[...] = bm.reshape(1, nq, _CHUNK // _BUCKET)


_KEY_LO = 1077936127


def _sc_select(sims2d, bmax, out, bmax_va, bmax_vb, ids_v, idx_va, idx_vb,
               cand_va, cand_vb, surv_v, orow_v, bsa, bsb, gsa, gsb, sem,
               *, nq, nb):
    wid = lax.axis_index("s") * _NC + lax.axis_index("c")
    rows_per_w = nq // _NW
    base = wid * rows_per_w
    nbv = nb // _L
    iota = lax.broadcasted_iota(jnp.int32, (_L,), 0)
    ninf = jnp.full((_L,), _NEG, jnp.float32)
    zf = jnp.zeros((_L,), jnp.float32)

    def popcnt(msk):
        return plsc.all_reduce_population_count(msk)

    def to_scalar_i(vec):
        return vec[0]

    def cnt_ge_static(ref, nvreg, t):
        cs = [jnp.zeros((_L,), jnp.int32) for _ in range(4)]
        for k in range(nvreg):
            cs[k % 4] = cs[k % 4] + popcnt(ref[pl.ds(k * _L, _L)] >= t)
        return (cs[0] + cs[1]) + (cs[2] + cs[3])

    def bis_while(cnt_fn, lo0, hi0):
        def cond(carry):
            lo, hi, it = carry
            return jnp.logical_and(it < 32, (hi - lo)[0] > 1)
        def body(carry):
            lo, hi, it = carry
            mid = lo + lax.shift_right_logical(hi - lo, jnp.uint32(1))
            ge = cnt_fn(_key2f(mid)) >= K_NN
            return jnp.where(ge, mid, lo), jnp.where(ge, hi, mid), it + 1
        lo, _, _ = lax.while_loop(cond, body, (lo0, hi0, jnp.int32(0)))
        return _key2f(lo)

    def a_phase(i, bmax_v, idx_v, bsem, gsem, cand_v):
        r = base + i
        pltpu.make_async_copy(bmax.at[r], bmax_v, bsem).wait()

        mxs = [ninf, ninf]
        mns = [jnp.full((_L,), jnp.inf, jnp.float32) for _ in range(2)]
        for k in range(nbv - 1):
            v = bmax_v[pl.ds(k * _L, _L)]
            mxs[k % 2] = jnp.maximum(mxs[k % 2], v)
            mns[k % 2] = jnp.minimum(mns[k % 2], v)
        mvec = jnp.maximum(jnp.maximum(mxs[0], mxs[1]),
                           bmax_v[pl.ds((nbv - 1) * _L, _L)])
        nvec = jnp.minimum(mns[0], mns[1])
        m = mvec[0]
        lo_f = nvec[0]
        for k in range(1, _L):
            m = jnp.maximum(m, mvec[k])
            lo_f = jnp.minimum(lo_f, nvec[k])
        m_v = jnp.full((_L,), 0.0, jnp.float32) + m
        lo_v = jnp.full((_L,), 0.0, jnp.float32) + lo_f

        t_low = bis_while(lambda t: cnt_ge_static(bmax_v, nbv, t),
                          _f2key(lo_v), _f2key(m_v) + jnp.uint32(1))

        def gt_body(k, off):
            v = bmax_v[pl.ds(k * _L, _L)]
            msk = v > t_low
            plsc.store_compressed(ids_v.at[pl.ds(off, _L)], k * _L + iota,
                                  mask=msk)
            return off + to_scalar_i(popcnt(msk))

        off = lax.fori_loop(0, nbv, gt_body, jnp.int32(0), unroll=4)

        def eq_body(k, off):
            v = bmax_v[pl.ds(k * _L, _L)]
            msk = v == t_low
            @pl.when(off < K_NN)
            def _():
                plsc.store_compressed(ids_v.at[pl.ds(off, _L)],
                                      k * _L + iota, mask=msk)
            return off + to_scalar_i(popcnt(msk))

        lax.fori_loop(0, nbv, eq_body, off, unroll=4)

        for k in range(K_NN // _L):
            idx_v[pl.ds(k * _L, _L)] = ids_v[pl.ds(k * _L, _L)] + r * nb
        pltpu.async_copy(sims2d.at[idx_v], cand_v, gsem)

        @pl.when(i + 2 < rows_per_w)
        def _():
            pltpu.async_copy(bmax.at[r + 2], bmax_v, bsem)
        return t_low, m_v

    def b_phase(i, cand_v, gsem, t_low, m_v):
        r = base + i
        pltpu.make_async_copy(sims2d.at[pl.ds(0, K_NN)], cand_v, gsem).wait()

        def f_body(k, offs):
            j2 = k >> 3
            lb = (k & 7) * _L
            new = []
            for g in range(4):
                v = cand_v[g * (K_NN // 4) + j2, pl.ds(lb, _L)]
                msk = v >= t_low
                plsc.store_compressed(surv_v.at[g, pl.ds(offs[g], _L)], v,
                                      mask=msk)
                new.append(offs[g] + to_scalar_i(popcnt(msk)))
            return tuple(new)

        z4 = (jnp.int32(0),) * 4
        offs = lax.fori_loop(0, (K_NN // 4) * (_BUCKET // _L), f_body, z4,
                             unroll=4)
        for g in range(4):
            surv_v[g, pl.ds(offs[g], _L)] = ninf
        nvs = [lax.div(offs[g] + (_L - 1), jnp.int32(_L)) for g in range(4)]

        def cnt4(t):
            c = jnp.zeros((_L,), jnp.int32)
            for g in range(4):
                def b(k, c2):
                    return c2 + popcnt(surv_v[g, pl.ds(k * _L, _L)] >= t)
                c = lax.fori_loop(0, nvs[g], b, c)
            return c

        v64 = bis_while(cnt4, _f2key(t_low), _f2key(m_v) + jnp.uint32(1))

        def s_body_g(g):
            def s_body(k, carry):
                zv, tv, cv = carry
                v = surv_v[g, pl.ds(k * _L, _L)]
                gt = v > v64
                e = jnp.exp(v - m_v)
                zv = zv + jnp.where(gt, e, 0.0)
                tv = tv + jnp.where(gt, (v - m_v) * e, 0.0)
                cv = cv + popcnt(gt)
                return zv, tv, cv
            return s_body

        carry = (zf, zf, jnp.zeros((_L,), jnp.int32))
        for g in range(4):
            carry = lax.fori_loop(0, nvs[g], s_body_g(g), carry)
        zv, tv, cv = carry
        z_gt = zv[0]
        t_gt = tv[0]
        for k in range(1, _L):
            z_gt = z_gt + zv[k]
            t_gt = t_gt + tv[k]

        orow_v[...] = ((iota == 0).astype(jnp.float32) * z_gt
                       + (iota == 1).astype(jnp.float32) * t_gt
                       + (iota == 2).astype(jnp.float32) * cv.astype(jnp.float32)
                       + (iota == 3).astype(jnp.float32) * v64
                       + (iota == 4).astype(jnp.float32) * m_v[0])
        pltpu.sync_copy(orow_v, out.at[r])

    pltpu.async_copy(bmax.at[base], bmax_va, bsa)
    pltpu.async_copy(bmax.at[base + 1], bmax_vb, bsb)

    def pair_body(r2, carry):
        t_prev, m_prev = carry
        t_a, m_a = a_phase(2 * r2, bmax_va, idx_va, bsa, gsa, cand_va)

        @pl.when(r2 >= 1)
        def _():
            b_phase(2 * r2 - 1, cand_vb, gsb, t_prev, m_prev)

        t_b, m_b = a_phase(2 * r2 + 1, bmax_vb, idx_vb, bsb, gsb, cand_vb)
        b_phase(2 * r2, cand_va, gsa, t_a, m_a)
        return t_b, m_b

    t_fin, m_fin = lax.fori_loop(0, rows_per_w // 2, pair_body, (zf, zf))
    b_phase(rows_per_w - 1, cand_vb, gsb, t_fin, m_fin)


def _finish_kernel(st_ref, out_ref, *, nq):
    x = st_ref[...]
    z_gt, t_gt = x[:, 0:1], x[:, 1:2]
    c_gt, v64, m = x[:, 2:3], x[:, 3:4], x[:, 4:5]
    n_tie = jnp.float32(K_NN) - c_gt
    ut = v64 - m
    et = jnp.exp(ut)
    z = z_gt + n_tie * et
    t = t_gt + n_tie * ut * et
    ent = jnp.log(z) - t / z
    out_ref[...] = jnp.sum(ent).reshape(1, 1) * (1.0 / nq)


def kernel(feat, gallery_features):
    nq, d = feat.shape
    ng = gallery_features.shape[0]
    ng_pad = ((ng + _CHUNK - 1) // _CHUNK) * _CHUNK
    nb = ng_pad // _BUCKET
    nchunks = ng_pad // _CHUNK
    bpc = _CHUNK // _BUCKET
    gal = jnp.pad(gallery_features, ((0, ng_pad - ng), (0, 0)))

    sims, bmax3 = pl.pallas_call(
        functools.partial(_sims_kernel, n_real=ng),
        grid=(nchunks,),
        in_specs=[
            pl.BlockSpec((nq, d), lambda j: (0, 0)),
            pl.BlockSpec((_CHUNK, d), lambda j: (j, 0)),
        ],
        out_specs=[
            pl.BlockSpec((nq, _CHUNK), lambda j: (0, j)),
            pl.BlockSpec((1, nq, bpc), lambda j: (j, 0, 0)),
        ],
        out_shape=[
            jax.ShapeDtypeStruct((nq, ng_pad), jnp.float32),
            jax.ShapeDtypeStruct((nchunks, nq, bpc), jnp.float32),
        ],
    )(feat, gal)

    bmax = bmax3.transpose(1, 0, 2).reshape(nq, nb)
    sims2d = sims.reshape(nq * nb, _BUCKET)

    sc_fn = functools.partial(
        pl.kernel,
        mesh=plsc.VectorSubcoreMesh(core_axis_name="c", subcore_axis_name="s"),
        compiler_params=pltpu.CompilerParams(needs_layout_passes=False),
        out_type=jax.ShapeDtypeStruct((nq, _L), jnp.float32),
        scratch_types=[
            pltpu.VMEM((nb,), jnp.float32),
            pltpu.VMEM((nb,), jnp.float32),
            pltpu.VMEM((K_NN + 2 * _L,), jnp.int32),
            pltpu.VMEM((K_NN,), jnp.int32),
            pltpu.VMEM((K_NN,), jnp.int32),
            pltpu.VMEM((K_NN, _BUCKET), jnp.float32),
            pltpu.VMEM((K_NN, _BUCKET), jnp.float32),
            pltpu.VMEM((4, K_NN * _BUCKET // 4 + _L), jnp.float32),
            pltpu.VMEM((_L,), jnp.float32),
            pltpu.SemaphoreType.DMA,
            pltpu.SemaphoreType.DMA,
            pltpu.SemaphoreType.DMA,
            pltpu.SemaphoreType.DMA,
            pltpu.SemaphoreType.DMA,
        ],
    )(functools.partial(_sc_select, nq=nq, nb=nb))
    stats = sc_fn(sims2d, bmax)

    out = pl.pallas_call(
        functools.partial(_finish_kernel, nq=float(nq)),
        in_specs=[pl.BlockSpec((nq, _L), lambda: (0, 0))],
        out_specs=pl.BlockSpec((1, 1), lambda: (0, 0)),
        out_shape=jax.ShapeDtypeStruct((1, 1), jnp.float32),
    )(stats)
    return out[0, 0]


# speedup vs baseline: 17.6913x; 1.0001x over previous
import functools

import jax
import jax.numpy as jnp
from jax import lax
from jax.experimental import pallas as pl
from jax.experimental.pallas import tpu as pltpu
from jax.experimental.pallas import tpu_sc as plsc

K_NN = 64
_CHUNK = 2048
_BUCKET = 128
_NEG = float("-inf")
_NC, _NS, _L = 2, 16, 16
_NW = _NC * _NS


def _f2key(x):
    b = lax.bitcast_convert_type(x, jnp.int32)
    mask = lax.shift_right_arithmetic(b, 31) & jnp.int32(0x7FFFFFFF)
    s = lax.bitwise_xor(b, mask)
    return lax.bitcast_convert_type(s, jnp.uint32) + jnp.uint32(0x80000000)


def _key2f(u):
    s = lax.bitcast_convert_type(u + jnp.uint32(0x80000000), jnp.int32)
    mask = lax.shift_right_arithmetic(s, 31) & jnp.int32(0x7FFFFFFF)
    b = lax.bitwise_xor(s, mask)
    return lax.bitcast_convert_type(b, jnp.float32)


def _sims_kernel(feat_ref, gal_ref, sims_ref, bmax_ref, *, n_real):
    j = pl.program_id(0)
    q = feat_ref[...]
    qn = q * lax.rsqrt(jnp.maximum(jnp.sum(q * q, axis=1, keepdims=True), 1e-30))
    g = gal_ref[...]
    gn = g * lax.rsqrt(jnp.maximum(jnp.sum(g * g, axis=1, keepdims=True), 1e-30))
    s = lax.dot_general(qn, gn, (((1,), (1,)), ((), ())),
                        preferred_element_type=jnp.float32)
    col = j * _CHUNK + lax.broadcasted_iota(jnp.int32, s.shape, 1)
    s = jnp.where(col < n_real, s, _NEG)
    sims_ref[...] = s
    nq = s.shape[0]
    bm = jnp.max(s.reshape(nq, _CHUNK // _BUCKET, _BUCKET), axis=2)
    bmax_ref[...] = bm.reshape(1, nq, _CHUNK // _BUCKET)


_KEY_LO = 1077936127


def _sc_select(sims2d, bmax, out, bmax_va, bmax_vb, ids_v, idx_va, idx_vb,
               cand_va, cand_vb, surv_v, orow_v, bsa, bsb, gsa, gsb, sem,
               *, nq, nb):
    wid = lax.axis_index("s") * _NC + lax.axis_index("c")
    rows_per_w = nq // _NW
    base = wid * rows_per_w
    nbv = nb // _L
    iota = lax.broadcasted_iota(jnp.int32, (_L,), 0)
    ninf = jnp.full((_L,), _NEG, jnp.float32)
    zf = jnp.zeros((_L,), jnp.float32)

    def popcnt(msk):
        return plsc.all_reduce_population_count(msk)

    def to_scalar_i(vec):
        return vec[0]

    def cnt_ge_static(ref, nvreg, t):
        cs = [jnp.zeros((_L,), jnp.int32) for _ in range(4)]
        for k in range(nvreg):
            cs[k % 4] = cs[k % 4] + popcnt(ref[pl.ds(k * _L, _L)] >= t)
        return (cs[0] + cs[1]) + (cs[2] + cs[3])

    def bis_while(cnt_fn, lo0, hi0):
        def cond(carry):
            lo, hi, it = carry
            return jnp.logical_and(it < 32, (hi - lo)[0] > 1)
        def body(carry):
            lo, hi, it = carry
            mid = lo + lax.shift_right_logical(hi - lo, jnp.uint32(1))
            ge = cnt_fn(_key2f(mid)) >= K_NN
            return jnp.where(ge, mid, lo), jnp.where(ge, hi, mid), it + 1
        lo, _, _ = lax.while_loop(cond, body, (lo0, hi0, jnp.int32(0)))
        return _key2f(lo)

    def a_phase(i, bmax_v, idx_v, bsem, gsem, cand_v):
        r = base + i
        pltpu.make_async_copy(bmax.at[r], bmax_v, bsem).wait()

        scope_a = jax.named_scope("a_bisect"); scope_a.__enter__()
        mxs = [ninf, ninf]
        mns = [jnp.full((_L,), jnp.inf, jnp.float32) for _ in range(2)]
        for k in range(nbv - 1):
            v = bmax_v[pl.ds(k * _L, _L)]
            mxs[k % 2] = jnp.maximum(mxs[k % 2], v)
            mns[k % 2] = jnp.minimum(mns[k % 2], v)
        mvec = jnp.maximum(jnp.maximum(mxs[0], mxs[1]),
                           bmax_v[pl.ds((nbv - 1) * _L, _L)])
        nvec = jnp.minimum(mns[0], mns[1])
        m = mvec[0]
        lo_f = nvec[0]
        for k in range(1, _L):
            m = jnp.maximum(m, mvec[k])
            lo_f = jnp.minimum(lo_f, nvec[k])
        m_v = jnp.full((_L,), 0.0, jnp.float32) + m
        lo_v = jnp.full((_L,), 0.0, jnp.float32) + lo_f

        t_low = bis_while(lambda t: cnt_ge_static(bmax_v, nbv, t),
                          _f2key(lo_v), _f2key(m_v) + jnp.uint32(1))

        scope_a.__exit__(None, None, None)
        scope_i = jax.named_scope("a_ids"); scope_i.__enter__()
        def gt_body(k, off):
            v = bmax_v[pl.ds(k * _L, _L)]
            msk = v > t_low
            plsc.store_compressed(ids_v.at[pl.ds(off, _L)], k * _L + iota,
                                  mask=msk)
            return off + to_scalar_i(popcnt(msk))

        off = lax.fori_loop(0, nbv, gt_body, jnp.int32(0), unroll=4)

        def eq_body(k, off):
            v = bmax_v[pl.ds(k * _L, _L)]
            msk = v == t_low
            @pl.when(off < K_NN)
            def _():
                plsc.store_compressed(ids_v.at[pl.ds(off, _L)],
                                      k * _L + iota, mask=msk)
            return off + to_scalar_i(popcnt(msk))

        lax.fori_loop(0, nbv, eq_body, off, unroll=4)

        for k in range(K_NN // _L):
            idx_v[pl.ds(k * _L, _L)] = ids_v[pl.ds(k * _L, _L)] + r * nb
        pltpu.async_copy(sims2d.at[idx_v], cand_v, gsem)

        scope_i.__exit__(None, None, None)
        @pl.when(i + 2 < rows_per_w)
        def _():
            pltpu.async_copy(bmax.at[r + 2], bmax_v, bsem)
        return t_low, m_v

    def b_phase(i, cand_v, gsem, t_low, m_v):
        r = base + i
        pltpu.make_async_copy(sims2d.at[pl.ds(0, K_NN)], cand_v, gsem).wait()

        scope_f = jax.named_scope("b_filter"); scope_f.__enter__()
        def f_body(k, offs):
            j2 = k >> 3
            lb = (k & 7) * _L
            new = []
            for g in range(4):
                v = cand_v[g * (K_NN // 4) + j2, pl.ds(lb, _L)]
                msk = v >= t_low
                plsc.store_compressed(surv_v.at[g, pl.ds(offs[g], _L)], v,
                                      mask=msk)
                new.append(offs[g] + to_scalar_i(popcnt(msk)))
            return tuple(new)

        z4 = (jnp.int32(0),) * 4
        offs = lax.fori_loop(0, (K_NN // 4) * (_BUCKET // _L), f_body, z4,
                             unroll=4)
        for g in range(4):
            surv_v[g, pl.ds(offs[g], _L)] = ninf
        nvs = [lax.div(offs[g] + (_L - 1), jnp.int32(_L)) for g in range(4)]

        scope_f.__exit__(None, None, None)
        scope_b2 = jax.named_scope("b_bisect2"); scope_b2.__enter__()
        def cnt4(t):
            c = jnp.zeros((_L,), jnp.int32)
            for g in range(4):
                def b(k, c2):
                    return c2 + popcnt(surv_v[g, pl.ds(k * _L, _L)] >= t)
                c = lax.fori_loop(0, nvs[g], b, c)
            return c

        v64 = bis_while(cnt4, _f2key(t_low), _f2key(m_v) + jnp.uint32(1))

        scope_b2.__exit__(None, None, None)
        scope_s = jax.named_scope("b_stats"); scope_s.__enter__()
        def s_body_g(g):
            def s_body(k, carry):
                zv, tv, cv = carry
                v = surv_v[g, pl.ds(k * _L, _L)]
                gt = v > v64
                e = jnp.exp(v - m_v)
                zv = zv + jnp.where(gt, e, 0.0)
                tv = tv + jnp.where(gt, (v - m_v) * e, 0.0)
                cv = cv + popcnt(gt)
                return zv, tv, cv
            return s_body

        carry = (zf, zf, jnp.zeros((_L,), jnp.int32))
        for g in range(4):
            carry = lax.fori_loop(0, nvs[g], s_body_g(g), carry)
        zv, tv, cv = carry
        z_gt = zv[0]
        t_gt = tv[0]
        for k in range(1, _L):
            z_gt = z_gt + zv[k]
            t_gt = t_gt + tv[k]

        orow_v[...] = ((iota == 0).astype(jnp.float32) * z_gt
                       + (iota == 1).astype(jnp.float32) * t_gt
                       + (iota == 2).astype(jnp.float32) * cv.astype(jnp.float32)
                       + (iota == 3).astype(jnp.float32) * v64
                       + (iota == 4).astype(jnp.float32) * m_v[0])
        scope_s.__exit__(None, None, None)
        pltpu.sync_copy(orow_v, out.at[r])

    pltpu.async_copy(bmax.at[base], bmax_va, bsa)
    pltpu.async_copy(bmax.at[base + 1], bmax_vb, bsb)

    def pair_body(r2, carry):
        t_prev, m_prev = carry
        t_a, m_a = a_phase(2 * r2, bmax_va, idx_va, bsa, gsa, cand_va)

        @pl.when(r2 >= 1)
        def _():
            b_phase(2 * r2 - 1, cand_vb, gsb, t_prev, m_prev)

        t_b, m_b = a_phase(2 * r2 + 1, bmax_vb, idx_vb, bsb, gsb, cand_vb)
        b_phase(2 * r2, cand_va, gsa, t_a, m_a)
        return t_b, m_b

    t_fin, m_fin = lax.fori_loop(0, rows_per_w // 2, pair_body, (zf, zf))
    b_phase(rows_per_w - 1, cand_vb, gsb, t_fin, m_fin)


def _finish_kernel(st_ref, out_ref, *, nq):
    x = st_ref[...]
    z_gt, t_gt = x[:, 0:1], x[:, 1:2]
    c_gt, v64, m = x[:, 2:3], x[:, 3:4], x[:, 4:5]
    n_tie = jnp.float32(K_NN) - c_gt
    ut = v64 - m
    et = jnp.exp(ut)
    z = z_gt + n_tie * et
    t = t_gt + n_tie * ut * et
    ent = jnp.log(z) - t / z
    out_ref[...] = jnp.sum(ent).reshape(1, 1) * (1.0 / nq)


def kernel(feat, gallery_features):
    nq, d = feat.shape
    ng = gallery_features.shape[0]
    ng_pad = ((ng + _CHUNK - 1) // _CHUNK) * _CHUNK
    nb = ng_pad // _BUCKET
    nchunks = ng_pad // _CHUNK
    bpc = _CHUNK // _BUCKET
    gal = jnp.pad(gallery_features, ((0, ng_pad - ng), (0, 0)))

    sims, bmax3 = pl.pallas_call(
        functools.partial(_sims_kernel, n_real=ng),
        grid=(nchunks,),
        in_specs=[
            pl.BlockSpec((nq, d), lambda j: (0, 0)),
            pl.BlockSpec((_CHUNK, d), lambda j: (j, 0)),
        ],
        out_specs=[
            pl.BlockSpec((nq, _CHUNK), lambda j: (0, j)),
            pl.BlockSpec((1, nq, bpc), lambda j: (j, 0, 0)),
        ],
        out_shape=[
            jax.ShapeDtypeStruct((nq, ng_pad), jnp.float32),
            jax.ShapeDtypeStruct((nchunks, nq, bpc), jnp.float32),
        ],
    )(feat, gal)

    bmax = bmax3.transpose(1, 0, 2).reshape(nq, nb)
    sims2d = sims.reshape(nq * nb, _BUCKET)

    sc_fn = functools.partial(
        pl.kernel,
        mesh=plsc.VectorSubcoreMesh(core_axis_name="c", subcore_axis_name="s"),
        compiler_params=pltpu.CompilerParams(needs_layout_passes=False),
        out_type=jax.ShapeDtypeStruct((nq, _L), jnp.float32),
        scratch_types=[
            pltpu.VMEM((nb,), jnp.float32),
            pltpu.VMEM((nb,), jnp.float32),
            pltpu.VMEM((K_NN + 2 * _L,), jnp.int32),
            pltpu.VMEM((K_NN,), jnp.int32),
            pltpu.VMEM((K_NN,), jnp.int32),
            pltpu.VMEM((K_NN, _BUCKET), jnp.float32),
            pltpu.VMEM((K_NN, _BUCKET), jnp.float32),
            pltpu.VMEM((4, K_NN * _BUCKET // 4 + _L), jnp.float32),
            pltpu.VMEM((_L,), jnp.float32),
            pltpu.SemaphoreType.DMA,
            pltpu.SemaphoreType.DMA,
            pltpu.SemaphoreType.DMA,
            pltpu.SemaphoreType.DMA,
            pltpu.SemaphoreType.DMA,
        ],
    )(functools.partial(_sc_select, nq=nq, nb=nb))
    stats = sc_fn(sims2d, bmax)

    out = pl.pallas_call(
        functools.partial(_finish_kernel, nq=float(nq)),
        in_specs=[pl.BlockSpec((nq, _L), lambda: (0, 0))],
        out_specs=pl.BlockSpec((1, 1), lambda: (0, 0)),
        out_shape=jax.ShapeDtypeStruct((1, 1), jnp.float32),
    )(stats)
    return out[0, 0]


# speedup vs baseline: 17.7644x; 1.0041x over previous
import functools

import jax
import jax.numpy as jnp
from jax import lax
from jax.experimental import pallas as pl
from jax.experimental.pallas import tpu as pltpu
from jax.experimental.pallas import tpu_sc as plsc

K_NN = 64
_CHUNK = 2048
_BUCKET = 128
_NEG = float("-inf")
_NC, _NS, _L = 2, 16, 16
_NW = _NC * _NS


def _f2key(x):
    b = lax.bitcast_convert_type(x, jnp.int32)
    mask = lax.shift_right_arithmetic(b, 31) & jnp.int32(0x7FFFFFFF)
    s = lax.bitwise_xor(b, mask)
    return lax.bitcast_convert_type(s, jnp.uint32) + jnp.uint32(0x80000000)


def _key2f(u):
    s = lax.bitcast_convert_type(u + jnp.uint32(0x80000000), jnp.int32)
    mask = lax.shift_right_arithmetic(s, 31) & jnp.int32(0x7FFFFFFF)
    b = lax.bitwise_xor(s, mask)
    return lax.bitcast_convert_type(b, jnp.float32)


def _sims_kernel(feat_ref, gal_ref, sims_ref, bmax_ref, *, n_real):
    j = pl.program_id(0)
    q = feat_ref[...]
    qn = q * lax.rsqrt(jnp.maximum(jnp.sum(q * q, axis=1, keepdims=True), 1e-30))
    g = gal_ref[...]
    gn = g * lax.rsqrt(jnp.maximum(jnp.sum(g * g, axis=1, keepdims=True), 1e-30))
    s = lax.dot_general(qn.astype(jnp.bfloat16), gn.astype(jnp.bfloat16),
                        (((1,), (1,)), ((), ())),
                        preferred_element_type=jnp.float32)
    col = j * _CHUNK + lax.broadcasted_iota(jnp.int32, s.shape, 1)
    s = jnp.where(col < n_real, s, _NEG)
    sims_ref[...] = s
    nq = s.shape[0]
    bm = jnp.max(s.reshape(nq, _CHUNK // _BUCKET, _BUCKET), axis=2)
    bmax_ref[...] = bm.reshape(1, nq, _CHUNK // _BUCKET)


_KEY_LO = 1077936127


def _sc_select(sims2d, bmax, out, bmax_va, bmax_vb, ids_v, idx_va, idx_vb,
               cand_va, cand_vb, surv_v, orow_v, bsa, bsb, gsa, gsb, sem,
               *, nq, nb):
    wid = lax.axis_index("s") * _NC + lax.axis_index("c")
    rows_per_w = nq // _NW
    base = wid * rows_per_w
    nbv = nb // _L
    iota = lax.broadcasted_iota(jnp.int32, (_L,), 0)
    ninf = jnp.full((_L,), _NEG, jnp.float32)
    zf = jnp.zeros((_L,), jnp.float32)

    def popcnt(msk):
        return plsc.all_reduce_population_count(msk)

    def to_scalar_i(vec):
        return vec[0]

    def cnt_ge_static(ref, nvreg, t):
        cs = [jnp.zeros((_L,), jnp.int32) for _ in range(4)]
        for k in range(nvreg):
            cs[k % 4] = cs[k % 4] + popcnt(ref[pl.ds(k * _L, _L)] >= t)
        return (cs[0] + cs[1]) + (cs[2] + cs[3])

    def bis_while(cnt_fn, lo0, hi0):
        def cond(carry):
            lo, hi, it = carry
            return jnp.logical_and(it < 32, (hi - lo)[0] > 1)
        def body(carry):
            lo, hi, it = carry
            mid = lo + lax.shift_right_logical(hi - lo, jnp.uint32(1))
            ge = cnt_fn(_key2f(mid)) >= K_NN
            return jnp.where(ge, mid, lo), jnp.where(ge, hi, mid), it + 1
        lo, _, _ = lax.while_loop(cond, body, (lo0, hi0, jnp.int32(0)))
        return _key2f(lo)

    def a_phase(i, bmax_v, idx_v, bsem, gsem, cand_v):
        r = base + i
        pltpu.make_async_copy(bmax.at[r], bmax_v, bsem).wait()

        mxs = [ninf, ninf]
        mns = [jnp.full((_L,), jnp.inf, jnp.float32) for _ in range(2)]
        for k in range(nbv - 1):
            v = bmax_v[pl.ds(k * _L, _L)]
            mxs[k % 2] = jnp.maximum(mxs[k % 2], v)
            mns[k % 2] = jnp.minimum(mns[k % 2], v)
        mvec = jnp.maximum(jnp.maximum(mxs[0], mxs[1]),
                           bmax_v[pl.ds((nbv - 1) * _L, _L)])
        nvec = jnp.minimum(mns[0], mns[1])
        m = mvec[0]
        lo_f = nvec[0]
        for k in range(1, _L):
            m = jnp.maximum(m, mvec[k])
            lo_f = jnp.minimum(lo_f, nvec[k])
        m_v = jnp.full((_L,), 0.0, jnp.float32) + m
        lo_v = jnp.full((_L,), 0.0, jnp.float32) + lo_f

        t_low = bis_while(lambda t: cnt_ge_static(bmax_v, nbv, t),
                          _f2key(lo_v), _f2key(m_v) + jnp.uint32(1))

        def gt_body(k, off):
            v = bmax_v[pl.ds(k * _L, _L)]
            msk = v > t_low
            plsc.store_compressed(ids_v.at[pl.ds(off, _L)], k * _L + iota,
                                  mask=msk)
            return off + to_scalar_i(popcnt(msk))

        off = lax.fori_loop(0, nbv, gt_body, jnp.int32(0), unroll=4)

        def eq_body(k, off):
            v = bmax_v[pl.ds(k * _L, _L)]
            msk = v == t_low
            @pl.when(off < K_NN)
            def _():
                plsc.store_compressed(ids_v.at[pl.ds(off, _L)],
                                      k * _L + iota, mask=msk)
            return off + to_scalar_i(popcnt(msk))

        lax.fori_loop(0, nbv, eq_body, off, unroll=4)

        for k in range(K_NN // _L):
            idx_v[pl.ds(k * _L, _L)] = ids_v[pl.ds(k * _L, _L)] + r * nb
        pltpu.async_copy(sims2d.at[idx_v], cand_v, gsem)

        @pl.when(i + 2 < rows_per_w)
        def _():
            pltpu.async_copy(bmax.at[r + 2], bmax_v, bsem)
        return t_low, m_v

    def b_phase(i, cand_v, gsem, t_low, m_v):
        r = base + i
        pltpu.make_async_copy(sims2d.at[pl.ds(0, K_NN)], cand_v, gsem).wait()

        def f_body(k, offs):
            j2 = k >> 3
            lb = (k & 7) * _L
            new = []
            for g in range(4):
                v = cand_v[g * (K_NN // 4) + j2, pl.ds(lb, _L)]
                msk = v >= t_low
                plsc.store_compressed(surv_v.at[g, pl.ds(offs[g], _L)], v,
                                      mask=msk)
                new.append(offs[g] + to_scalar_i(popcnt(msk)))
            return tuple(new)

        z4 = (jnp.int32(0),) * 4
        offs = lax.fori_loop(0, (K_NN // 4) * (_BUCKET // _L), f_body, z4,
                             unroll=4)
        for g in range(4):
            surv_v[g, pl.ds(offs[g], _L)] = ninf
        nvs = [lax.div(offs[g] + (_L - 1), jnp.int32(_L)) for g in range(4)]

        def cnt4(t):
            c = jnp.zeros((_L,), jnp.int32)
            for g in range(4):
                def b(k, c2):
                    return c2 + popcnt(surv_v[g, pl.ds(k * _L, _L)] >= t)
                c = lax.fori_loop(0, nvs[g], b, c)
            return c

        v64 = bis_while(cnt4, _f2key(t_low), _f2key(m_v) + jnp.uint32(1))

        def s_body_g(g):
            def s_body(k, carry):
                zv, tv, cv = carry
                v = surv_v[g, pl.ds(k * _L, _L)]
                gt = v > v64
                e = jnp.exp(v - m_v)
                zv = zv + jnp.where(gt, e, 0.0)
                tv = tv + jnp.where(gt, (v - m_v) * e, 0.0)
                cv = cv + popcnt(gt)
                return zv, tv, cv
            return s_body

        carry = (zf, zf, jnp.zeros((_L,), jnp.int32))
        for g in range(4):
            carry = lax.fori_loop(0, nvs[g], s_body_g(g), carry)
        zv, tv, cv = carry
        z_gt = zv[0]
        t_gt = tv[0]
        for k in range(1, _L):
            z_gt = z_gt + zv[k]
            t_gt = t_gt + tv[k]

        orow_v[...] = ((iota == 0).astype(jnp.float32) * z_gt
                       + (iota == 1).astype(jnp.float32) * t_gt
                       + (iota == 2).astype(jnp.float32) * cv.astype(jnp.float32)
                       + (iota == 3).astype(jnp.float32) * v64
                       + (iota == 4).astype(jnp.float32) * m_v[0])
        pltpu.sync_copy(orow_v, out.at[r])

    pltpu.async_copy(bmax.at[base], bmax_va, bsa)
    pltpu.async_copy(bmax.at[base + 1], bmax_vb, bsb)

    def pair_body(r2, carry):
        t_prev, m_prev = carry
        t_a, m_a = a_phase(2 * r2, bmax_va, idx_va, bsa, gsa, cand_va)

        @pl.when(r2 >= 1)
        def _():
            b_phase(2 * r2 - 1, cand_vb, gsb, t_prev, m_prev)

        t_b, m_b = a_phase(2 * r2 + 1, bmax_vb, idx_vb, bsb, gsb, cand_vb)
        b_phase(2 * r2, cand_va, gsa, t_a, m_a)
        return t_b, m_b

    t_fin, m_fin = lax.fori_loop(0, rows_per_w // 2, pair_body, (zf, zf))
    b_phase(rows_per_w - 1, cand_vb, gsb, t_fin, m_fin)


def _finish_kernel(st_ref, out_ref, *, nq):
    x = st_ref[...]
    z_gt, t_gt = x[:, 0:1], x[:, 1:2]
    c_gt, v64, m = x[:, 2:3], x[:, 3:4], x[:, 4:5]
    n_tie = jnp.float32(K_NN) - c_gt
    ut = v64 - m
    et = jnp.exp(ut)
    z = z_gt + n_tie * et
    t = t_gt + n_tie * ut * et
    ent = jnp.log(z) - t / z
    out_ref[...] = jnp.sum(ent).reshape(1, 1) * (1.0 / nq)


def kernel(feat, gallery_features):
    nq, d = feat.shape
    ng = gallery_features.shape[0]
    ng_pad = ((ng + _CHUNK - 1) // _CHUNK) * _CHUNK
    nb = ng_pad // _BUCKET
    nchunks = ng_pad // _CHUNK
    bpc = _CHUNK // _BUCKET
    gal = jnp.pad(gallery_features, ((0, ng_pad - ng), (0, 0)))

    sims, bmax3 = pl.pallas_call(
        functools.partial(_sims_kernel, n_real=ng),
        grid=(nchunks,),
        in_specs=[
            pl.BlockSpec((nq, d), lambda j: (0, 0)),
            pl.BlockSpec((_CHUNK, d), lambda j: (j, 0)),
        ],
        out_specs=[
            pl.BlockSpec((nq, _CHUNK), lambda j: (0, j)),
            pl.BlockSpec((1, nq, bpc), lambda j: (j, 0, 0)),
        ],
        out_shape=[
            jax.ShapeDtypeStruct((nq, ng_pad), jnp.float32),
            jax.ShapeDtypeStruct((nchunks, nq, bpc), jnp.float32),
        ],
    )(feat, gal)

    bmax = bmax3.transpose(1, 0, 2).reshape(nq, nb)
    sims2d = sims.reshape(nq * nb, _BUCKET)

    sc_fn = functools.partial(
        pl.kernel,
        mesh=plsc.VectorSubcoreMesh(core_axis_name="c", subcore_axis_name="s"),
        compiler_params=pltpu.CompilerParams(needs_layout_passes=False),
        out_type=jax.ShapeDtypeStruct((nq, _L), jnp.float32),
        scratch_types=[
            pltpu.VMEM((nb,), jnp.float32),
            pltpu.VMEM((nb,), jnp.float32),
            pltpu.VMEM((K_NN + 2 * _L,), jnp.int32),
            pltpu.VMEM((K_NN,), jnp.int32),
            pltpu.VMEM((K_NN,), jnp.int32),
            pltpu.VMEM((K_NN, _BUCKET), jnp.float32),
            pltpu.VMEM((K_NN, _BUCKET), jnp.float32),
            pltpu.VMEM((4, K_NN * _BUCKET // 4 + _L), jnp.float32),
            pltpu.VMEM((_L,), jnp.float32),
            pltpu.SemaphoreType.DMA,
            pltpu.SemaphoreType.DMA,
            pltpu.SemaphoreType.DMA,
            pltpu.SemaphoreType.DMA,
            pltpu.SemaphoreType.DMA,
        ],
    )(functools.partial(_sc_select, nq=nq, nb=nb))
    stats = sc_fn(sims2d, bmax)

    out = pl.pallas_call(
        functools.partial(_finish_kernel, nq=float(nq)),
        in_specs=[pl.BlockSpec((nq, _L), lambda: (0, 0))],
        out_specs=pl.BlockSpec((1, 1), lambda: (0, 0)),
        out_shape=jax.ShapeDtypeStruct((1, 1), jnp.float32),
    )(stats)
    return out[0, 0]


# speedup vs baseline: 18.0118x; 1.0139x over previous
import functools

import jax
import jax.numpy as jnp
from jax import lax
from jax.experimental import pallas as pl
from jax.experimental.pallas import tpu as pltpu
from jax.experimental.pallas import tpu_sc as plsc

K_NN = 64
_CHUNK = 2048
_BUCKET = 256
_PK = _BUCKET // 2
_BPC = _CHUNK // _BUCKET
_NEG = float("-inf")
_NC, _NS, _L = 2, 16, 16
_NW = _NC * _NS


def _f2key(x):
    b = lax.bitcast_convert_type(x, jnp.int32)
    mask = lax.shift_right_arithmetic(b, 31) & jnp.int32(0x7FFFFFFF)
    s = lax.bitwise_xor(b, mask)
    return lax.bitcast_convert_type(s, jnp.uint32) + jnp.uint32(0x80000000)


def _key2f(u):
    s = lax.bitcast_convert_type(u + jnp.uint32(0x80000000), jnp.int32)
    mask = lax.shift_right_arithmetic(s, 31) & jnp.int32(0x7FFFFFFF)
    b = lax.bitwise_xor(s, mask)
    return lax.bitcast_convert_type(b, jnp.float32)


def _sims_kernel(feat_ref, glo_ref, ghi_ref, pk_ref, bmax_ref, *, n_real):
    j = pl.program_id(0)
    q = feat_ref[...]
    qn = (q * lax.rsqrt(jnp.maximum(jnp.sum(q * q, axis=1, keepdims=True),
                                    1e-30))).astype(jnp.bfloat16)
    nq = q.shape[0]

    def half(g_ref, col0):
        g = g_ref[...].reshape(_CHUNK // 2, q.shape[1])
        gn = (g * lax.rsqrt(jnp.maximum(jnp.sum(g * g, axis=1, keepdims=True),
                                        1e-30))).astype(jnp.bfloat16)
        s = lax.dot_general(qn, gn, (((1,), (1,)), ((), ())),
                            preferred_element_type=jnp.float32)
        li = lax.broadcasted_iota(jnp.int32, s.shape, 1)
        col = col0 + lax.shift_right_logical(li, 7) * _BUCKET + (li & 127)
        s = jnp.where(col < n_real, s, _NEG)
        s16 = s.astype(jnp.bfloat16)
        bm = jnp.max(s16.astype(jnp.float32).reshape(nq, _BPC, _PK), axis=2)
        u = lax.bitcast_convert_type(s16, jnp.uint16).astype(jnp.uint32)
        return u, bm

    u_lo, bm_lo = half(glo_ref, j * _CHUNK)
    u_hi, bm_hi = half(ghi_ref, j * _CHUNK + _PK)
    pk = lax.shift_left(u_hi, jnp.uint32(16)) | u_lo
    pk_ref[...] = lax.bitcast_convert_type(pk, jnp.int32)
    bmax_ref[...] = jnp.maximum(bm_lo, bm_hi).reshape(1, nq, _BPC)


def _sc_select(sims2d, bmax, out, bmax_va, bmax_vb, ids_v, idx_va, idx_vb,
               cand_va, cand_vb, surv_v, orow_v, bsa, bsb, gsa, gsb, sem,
               *, nq, nb_rows, nb_pad):
    wid = lax.axis_index("s") * _NC + lax.axis_index("c")
    rows_per_w = nq // _NW
    base = wid * rows_per_w
    nbv = nb_pad // _L
    iota = lax.broadcasted_iota(jnp.int32, (_L,), 0)
    ninf = jnp.full((_L,), _NEG, jnp.float32)
    zf = jnp.zeros((_L,), jnp.float32)

    def popcnt(msk):
        return plsc.all_reduce_population_count(msk)

    def to_scalar_i(vec):
        return vec[0]

    def cnt_ge_static(ref, nvreg, t):
        cs = [jnp.zeros((_L,), jnp.int32) for _ in range(4)]
        for k in range(nvreg):
            cs[k % 4] = cs[k % 4] + popcnt(ref[pl.ds(k * _L, _L)] >= t)
        return (cs[0] + cs[1]) + (cs[2] + cs[3])

    def bis_while(cnt_fn, lo0, hi0):
        def cond(carry):
            lo, hi, it = carry
            return jnp.logical_and(it < 32, (hi - lo)[0] > 1)
        def body(carry):
            lo, hi, it = carry
            mid = lo + lax.shift_right_logical(hi - lo, jnp.uint32(1))
            ge = cnt_fn(_key2f(mid)) >= K_NN
            return jnp.where(ge, mid, lo), jnp.where(ge, hi, mid), it + 1
        lo, _, _ = lax.while_loop(cond, body, (lo0, hi0, jnp.int32(0)))
        return _key2f(lo)

    def a_phase(i, bmax_v, idx_v, bsem, gsem, cand_v):
        r = base + i
        pltpu.make_async_copy(bmax.at[r], bmax_v, bsem).wait()

        mxs = [ninf, ninf]
        mns = [jnp.full((_L,), jnp.inf, jnp.float32) for _ in range(2)]
        for k in range(nbv - 1):
            v = bmax_v[pl.ds(k * _L, _L)]
            mxs[k % 2] = jnp.maximum(mxs[k % 2], v)
            mns[k % 2] = jnp.minimum(mns[k % 2], v)
        mvec = jnp.maximum(jnp.maximum(mxs[0], mxs[1]),
                           bmax_v[pl.ds((nbv - 1) * _L, _L)])
        nvec = jnp.minimum(mns[0], mns[1])
        m = mvec[0]
        lo_f = nvec[0]
        for k in range(1, _L):
            m = jnp.maximum(m, mvec[k])
            lo_f = jnp.minimum(lo_f, nvec[k])
        m_v = jnp.full((_L,), 0.0, jnp.float32) + m
        lo_v = jnp.full((_L,), 0.0, jnp.float32) + lo_f

        t_low = bis_while(lambda t: cnt_ge_static(bmax_v, nbv, t),
                          _f2key(lo_v), _f2key(m_v) + jnp.uint32(1))

        def gt_body(k, off):
            v = bmax_v[pl.ds(k * _L, _L)]
            msk = v > t_low
            plsc.store_compressed(ids_v.at[pl.ds(off, _L)], k * _L + iota,
                                  mask=msk)
            return off + to_scalar_i(popcnt(msk))

        off = lax.fori_loop(0, nbv, gt_body, jnp.int32(0), unroll=4)

        def eq_body(k, off):
            v = bmax_v[pl.ds(k * _L, _L)]
            msk = v == t_low
            @pl.when(off < K_NN)
            def _():
                plsc.store_compressed(ids_v.at[pl.ds(off, _L)],
                                      k * _L + iota, mask=msk)
            return off + to_scalar_i(popcnt(msk))

        lax.fori_loop(0, nbv, eq_body, off, unroll=4)

        for k in range(K_NN // _L):
            idx_v[pl.ds(k * _L, _L)] = ids_v[pl.ds(k * _L, _L)] + r * nb_rows
        pltpu.async_copy(sims2d.at[idx_v], cand_v, gsem)

        @pl.when(i + 2 < rows_per_w)
        def _():
            pltpu.async_copy(bmax.at[r + 2], bmax_v, bsem)
        return t_low, m_v

    def b_phase(i, cand_v, gsem, t_low, m_v):
        r = base + i
        pltpu.make_async_copy(sims2d.at[pl.ds(0, K_NN)], cand_v, gsem).wait()

        himask = jnp.full((_L,), -65536, jnp.int32)
        def f_body(k, offs):
            j2 = k >> 3
            lb = (k & 7) * _L
            new = list(offs)
            for g in range(4):
                w = cand_v[g * (K_NN // 4) + j2, pl.ds(lb, _L)]
                f_lo = lax.bitcast_convert_type(
                    lax.shift_left(w, jnp.int32(16)), jnp.float32)
                f_hi = lax.bitcast_convert_type(w & himask, jnp.float32)
                for h, fv in ((0, f_lo), (1, f_hi)):
                    sg = 2 * g + h
                    msk = fv >= t_low
                    plsc.store_compressed(surv_v.at[sg, pl.ds(new[sg], _L)],
                                          fv, mask=msk)
                    new[sg] = new[sg] + to_scalar_i(popcnt(msk))
            return tuple(new)

        z8 = (jnp.int32(0),) * 8
        offs = lax.fori_loop(0, (K_NN // 4) * (_PK // _L), f_body, z8,
                             unroll=4)
        for g in range(8):
            surv_v[g, pl.ds(offs[g], _L)] = ninf
        nvs = [lax.div(offs[g] + (_L - 1), jnp.int32(_L)) for g in range(8)]

        def cnt8(t):
            c = jnp.zeros((_L,), jnp.int32)
            for g in range(8):
                def b(k, c2):
                    return c2 + popcnt(surv_v[g, pl.ds(k * _L, _L)] >= t)
                c = lax.fori_loop(0, nvs[g], b, c)
            return c

        v64 = bis_while(cnt8, _f2key(t_low), _f2key(m_v) + jnp.uint32(1))

        def s_body_g(g):
            def s_body(k, carry):
                zv, tv, cv = carry
                v = surv_v[g, pl.ds(k * _L, _L)]
                gt = v > v64
                e = jnp.exp(v - m_v)
                zv = zv + jnp.where(gt, e, 0.0)
                tv = tv + jnp.where(gt, (v - m_v) * e, 0.0)
                cv = cv + popcnt(gt)
                return zv, tv, cv
            return s_body

        carry = (zf, zf, jnp.zeros((_L,), jnp.int32))
        for g in range(8):
            carry = lax.fori_loop(0, nvs[g], s_body_g(g), carry)
        zv, tv, cv = carry
        z_gt = zv[0]
        t_gt = tv[0]
        for k in range(1, _L):
            z_gt = z_gt + zv[k]
            t_gt = t_gt + tv[k]

        orow_v[...] = ((iota == 0).astype(jnp.float32) * z_gt
                       + (iota == 1).astype(jnp.float32) * t_gt
                       + (iota == 2).astype(jnp.float32) * cv.astype(jnp.float32)
                       + (iota == 3).astype(jnp.float32) * v64
                       + (iota == 4).astype(jnp.float32) * m_v[0])
        pltpu.sync_copy(orow_v, out.at[r])

    pltpu.async_copy(bmax.at[base], bmax_va, bsa)
    pltpu.async_copy(bmax.at[base + 1], bmax_vb, bsb)

    def pair_body(r2, carry):
        t_prev, m_prev = carry
        t_a, m_a = a_phase(2 * r2, bmax_va, idx_va, bsa, gsa, cand_va)

        @pl.when(r2 >= 1)
        def _():
            b_phase(2 * r2 - 1, cand_vb, gsb, t_prev, m_prev)

        t_b, m_b = a_phase(2 * r2 + 1, bmax_vb, idx_vb, bsb, gsb, cand_vb)
        b_phase(2 * r2, cand_va, gsa, t_a, m_a)
        return t_b, m_b

    t_fin, m_fin = lax.fori_loop(0, rows_per_w // 2, pair_body, (zf, zf))
    b_phase(rows_per_w - 1, cand_vb, gsb, t_fin, m_fin)


def _finish_kernel(st_ref, out_ref, *, nq):
    x = st_ref[...]
    z_gt, t_gt = x[:, 0:1], x[:, 1:2]
    c_gt, v64, m = x[:, 2:3], x[:, 3:4], x[:, 4:5]
    n_tie = jnp.float32(K_NN) - c_gt
    ut = v64 - m
    et = jnp.exp(ut)
    z = z_gt + n_tie * et
    t = t_gt + n_tie * ut * et
    ent = jnp.log(z) - t / z
    out_ref[...] = jnp.sum(ent).reshape(1, 1) * (1.0 / nq)


def kernel(feat, gallery_features):
    nq, d = feat.shape
    ng = gallery_features.shape[0]
    ng_pad = ((ng + _CHUNK - 1) // _CHUNK) * _CHUNK
    nb_rows = ng_pad // _BUCKET
    nb_pad = ((nb_rows + _L - 1) // _L) * _L
    nchunks = ng_pad // _CHUNK
    gal = jnp.pad(gallery_features, ((0, ng_pad - ng), (0, 0)))
    gal4 = gal.reshape(nb_rows, 2, _PK, d)

    pk, bmax3 = pl.pallas_call(
        functools.partial(_sims_kernel, n_real=ng),
        grid=(nchunks,),
        in_specs=[
            pl.BlockSpec((nq, d), lambda j: (0, 0)),
            pl.BlockSpec((_BPC, 1, _PK, d), lambda j: (j, 0, 0, 0)),
            pl.BlockSpec((_BPC, 1, _PK, d), lambda j: (j, 1, 0, 0)),
        ],
        out_specs=[
            pl.BlockSpec((nq, _CHUNK // 2), lambda j: (0, j)),
            pl.BlockSpec((1, nq, _BPC), lambda j: (j, 0, 0)),
        ],
        out_shape=[
            jax.ShapeDtypeStruct((nq, ng_pad // 2), jnp.int32),
            jax.ShapeDtypeStruct((nchunks, nq, _BPC), jnp.float32),
        ],
    )(feat, gal4, gal4)

    bmax = jnp.pad(bmax3.transpose(1, 0, 2).reshape(nq, nb_rows),
                   ((0, 0), (0, nb_pad - nb_rows)), constant_values=_NEG)
    sims2d = pk.reshape(nq * nb_rows, _PK)

    sc_fn = functools.partial(
        pl.kernel,
        mesh=plsc.VectorSubcoreMesh(core_axis_name="c", subcore_axis_name="s"),
        compiler_params=pltpu.CompilerParams(needs_layout_passes=False),
        out_type=jax.ShapeDtypeStruct((nq, _L), jnp.float32),
        scratch_types=[
            pltpu.VMEM((nb_pad,), jnp.float32),
            pltpu.VMEM((nb_pad,), jnp.float32),
            pltpu.VMEM((K_NN + 2 * _L,), jnp.int32),
            pltpu.VMEM((K_NN,), jnp.int32),
            pltpu.VMEM((K_NN,), jnp.int32),
            pltpu.VMEM((K_NN, _PK), jnp.int32),
            pltpu.VMEM((K_NN, _PK), jnp.int32),
            pltpu.VMEM((8, K_NN * _PK // 4 + _L), jnp.float32),
            pltpu.VMEM((_L,), jnp.float32),
            pltpu.SemaphoreType.DMA,
            pltpu.SemaphoreType.DMA,
            pltpu.SemaphoreType.DMA,
            pltpu.SemaphoreType.DMA,
            pltpu.SemaphoreType.DMA,
        ],
    )(functools.partial(_sc_select, nq=nq, nb_rows=nb_rows, nb_pad=nb_pad))
    stats = sc_fn(sims2d, bmax)

    out = pl.pallas_call(
        functools.partial(_finish_kernel, nq=float(nq)),
        in_specs=[pl.BlockSpec((nq, _L), lambda: (0, 0))],
        out_specs=pl.BlockSpec((1, 1), lambda: (0, 0)),
        out_shape=jax.ShapeDtypeStruct((1, 1), jnp.float32),
    )(stats)
    return out[0, 0]


# speedup vs baseline: 21.0707x; 1.1698x over previous
import functools

import jax
import jax.numpy as jnp
from jax import lax
from jax.experimental import pallas as pl
from jax.experimental.pallas import tpu as pltpu
from jax.experimental.pallas import tpu_sc as plsc

K_NN = 64
_CHUNK = 2048
_BUCKET = 256
_PK = _BUCKET // 2
_BPC = _CHUNK // _BUCKET
_NEG = float("-inf")
_NC, _NS, _L = 2, 16, 16
_NW = _NC * _NS


def _f2key(x):
    b = lax.bitcast_convert_type(x, jnp.int32)
    mask = lax.shift_right_arithmetic(b, 31) & jnp.int32(0x7FFFFFFF)
    s = lax.bitwise_xor(b, mask)
    return lax.bitcast_convert_type(s, jnp.uint32) + jnp.uint32(0x80000000)


def _key2f(u):
    s = lax.bitcast_convert_type(u + jnp.uint32(0x80000000), jnp.int32)
    mask = lax.shift_right_arithmetic(s, 31) & jnp.int32(0x7FFFFFFF)
    b = lax.bitwise_xor(s, mask)
    return lax.bitcast_convert_type(b, jnp.float32)


def _sims_kernel(feat_ref, glo_ref, ghi_ref, pk_ref, bmax_ref, *, n_real):
    j = pl.program_id(0)
    q = feat_ref[...]
    qn = (q * lax.rsqrt(jnp.maximum(jnp.sum(q * q, axis=1, keepdims=True),
                                    1e-30))).astype(jnp.bfloat16)
    nq = q.shape[0]

    def norm16(g_ref):
        g = g_ref[...].reshape(_CHUNK // 2, q.shape[1])
        return (g * lax.rsqrt(jnp.maximum(
            jnp.sum(g * g, axis=1, keepdims=True), 1e-30))).astype(jnp.bfloat16)

    gn_lo = norm16(glo_ref)
    gn_hi = norm16(ghi_ref)
    li = lax.broadcasted_iota(jnp.int32, (nq, _PK), 1)
    bms = []
    for b in range(_BPC):
        def half(gn, col0):
            s = lax.dot_general(qn, gn[b * _PK:(b + 1) * _PK, :],
                                (((1,), (1,)), ((), ())),
                                preferred_element_type=jnp.float32)
            s = jnp.where(li + col0 < n_real, s, _NEG)
            s16 = s.astype(jnp.bfloat16)
            sf = s16.astype(jnp.float32)
            bm = jnp.max(sf, axis=1, keepdims=True)
            u = lax.bitcast_convert_type(s16, jnp.uint16).astype(jnp.uint32)
            return u, bm

        c0 = j * _CHUNK + b * _BUCKET
        u_lo, bm_lo = half(gn_lo, c0)
        u_hi, bm_hi = half(gn_hi, c0 + _PK)
        pk = lax.shift_left(u_hi, jnp.uint32(16)) | u_lo
        pk_ref[:, b, :] = lax.bitcast_convert_type(pk, jnp.int32)
        bms.append(jnp.maximum(bm_lo, bm_hi))
    bmax_ref[...] = jnp.concatenate(bms, axis=1).reshape(1, nq, _BPC)


def _sc_select(sims2d, bmax, out, bmax_va, bmax_vb, ids_v, idx_va, idx_vb,
               cand_va, cand_vb, surv_v, orow_v, bsa, bsb, gsa, gsb, sem,
               *, nq, nb_rows, nb_pad):
    wid = lax.axis_index("s") * _NC + lax.axis_index("c")
    rows_per_w = nq // _NW
    base = wid * rows_per_w
    nbv = nb_pad // _L
    iota = lax.broadcasted_iota(jnp.int32, (_L,), 0)
    ninf = jnp.full((_L,), _NEG, jnp.float32)
    zf = jnp.zeros((_L,), jnp.float32)

    def popcnt(msk):
        return plsc.all_reduce_population_count(msk)

    def to_scalar_i(vec):
        return vec[0]

    def cnt_ge_static(ref, nvreg, t):
        cs = [jnp.zeros((_L,), jnp.int32) for _ in range(4)]
        for k in range(nvreg):
            cs[k % 4] = cs[k % 4] + popcnt(ref[pl.ds(k * _L, _L)] >= t)
        return (cs[0] + cs[1]) + (cs[2] + cs[3])

    def bis_while(cnt_fn, lo0, hi0):
        def cond(carry):
            lo, hi, it = carry
            return jnp.logical_and(it < 32, (hi - lo)[0] > 1)
        def body(carry):
            lo, hi, it = carry
            mid = lo + lax.shift_right_logical(hi - lo, jnp.uint32(1))
            ge = cnt_fn(_key2f(mid)) >= K_NN
            return jnp.where(ge, mid, lo), jnp.where(ge, hi, mid), it + 1
        lo, _, _ = lax.while_loop(cond, body, (lo0, hi0, jnp.int32(0)))
        return _key2f(lo)

    def a_phase(i, bmax_v, idx_v, bsem, gsem, cand_v):
        r = base + i
        pltpu.make_async_copy(bmax.at[r], bmax_v, bsem).wait()

        mxs = [ninf, ninf]
        mns = [jnp.full((_L,), jnp.inf, jnp.float32) for _ in range(2)]
        for k in range(nbv - 1):
            v = bmax_v[pl.ds(k * _L, _L)]
            mxs[k % 2] = jnp.maximum(mxs[k % 2], v)
            mns[k % 2] = jnp.minimum(mns[k % 2], v)
        mvec = jnp.maximum(jnp.maximum(mxs[0], mxs[1]),
                           bmax_v[pl.ds((nbv - 1) * _L, _L)])
        nvec = jnp.minimum(mns[0], mns[1])
        m = mvec[0]
        lo_f = nvec[0]
        for k in range(1, _L):
            m = jnp.maximum(m, mvec[k])
            lo_f = jnp.minimum(lo_f, nvec[k])
        m_v = jnp.full((_L,), 0.0, jnp.float32) + m
        lo_v = jnp.full((_L,), 0.0, jnp.float32) + lo_f

        t_low = bis_while(lambda t: cnt_ge_static(bmax_v, nbv, t),
                          _f2key(lo_v), _f2key(m_v) + jnp.uint32(1))

        def gt_body(k, off):
            v = bmax_v[pl.ds(k * _L, _L)]
            msk = v > t_low
            plsc.store_compressed(ids_v.at[pl.ds(off, _L)], k * _L + iota,
                                  mask=msk)
            return off + to_scalar_i(popcnt(msk))

        off = lax.fori_loop(0, nbv, gt_body, jnp.int32(0), unroll=4)

        def eq_body(k, off):
            v = bmax_v[pl.ds(k * _L, _L)]
            msk = v == t_low
            @pl.when(off < K_NN)
            def _():
                plsc.store_compressed(ids_v.at[pl.ds(off, _L)],
                                      k * _L + iota, mask=msk)
            return off + to_scalar_i(popcnt(msk))

        lax.fori_loop(0, nbv, eq_body, off, unroll=4)

        for k in range(K_NN // _L):
            idx_v[pl.ds(k * _L, _L)] = ids_v[pl.ds(k * _L, _L)] + r * nb_rows
        pltpu.async_copy(sims2d.at[idx_v], cand_v, gsem)

        @pl.when(i + 2 < rows_per_w)
        def _():
            pltpu.async_copy(bmax.at[r + 2], bmax_v, bsem)
        return t_low, m_v

    def b_phase(i, cand_v, gsem, t_low, m_v):
        r = base + i
        pltpu.make_async_copy(sims2d.at[pl.ds(0, K_NN)], cand_v, gsem).wait()

        himask = jnp.full((_L,), -65536, jnp.int32)
        def f_body(k, offs):
            j2 = k >> 3
            lb = (k & 7) * _L
            new = list(offs)
            for g in range(4):
                w = cand_v[g * (K_NN // 4) + j2, pl.ds(lb, _L)]
                f_lo = lax.bitcast_convert_type(
                    lax.shift_left(w, jnp.int32(16)), jnp.float32)
                f_hi = lax.bitcast_convert_type(w & himask, jnp.float32)
                for h, fv in ((0, f_lo), (1, f_hi)):
                    sg = 2 * g + h
                    msk = fv >= t_low
                    plsc.store_compressed(surv_v.at[sg, pl.ds(new[sg], _L)],
                                          fv, mask=msk)
                    new[sg] = new[sg] + to_scalar_i(popcnt(msk))
            return tuple(new)

        z8 = (jnp.int32(0),) * 8
        offs = lax.fori_loop(0, (K_NN // 4) * (_PK // _L), f_body, z8,
                             unroll=4)
        for g in range(8):
            surv_v[g, pl.ds(offs[g], _L)] = ninf
        nvs = [lax.div(offs[g] + (_L - 1), jnp.int32(_L)) for g in range(8)]

        def cnt8(t):
            c = jnp.zeros((_L,), jnp.int32)
            for g in range(8):
                def b(k, c2):
                    return c2 + popcnt(surv_v[g, pl.ds(k * _L, _L)] >= t)
                c = lax.fori_loop(0, nvs[g], b, c)
            return c

        v64 = bis_while(cnt8, _f2key(t_low), _f2key(m_v) + jnp.uint32(1))

        def s_body_g(g):
            def s_body(k, carry):
                zv, tv, cv = carry
                v = surv_v[g, pl.ds(k * _L, _L)]
                gt = v > v64
                e = jnp.exp(v - m_v)
                zv = zv + jnp.where(gt, e, 0.0)
                tv = tv + jnp.where(gt, (v - m_v) * e, 0.0)
                cv = cv + popcnt(gt)
                return zv, tv, cv
            return s_body

        carry = (zf, zf, jnp.zeros((_L,), jnp.int32))
        for g in range(8):
            carry = lax.fori_loop(0, nvs[g], s_body_g(g), carry)
        zv, tv, cv = carry
        z_gt = zv[0]
        t_gt = tv[0]
        for k in range(1, _L):
            z_gt = z_gt + zv[k]
            t_gt = t_gt + tv[k]

        orow_v[...] = ((iota == 0).astype(jnp.float32) * z_gt
                       + (iota == 1).astype(jnp.float32) * t_gt
                       + (iota == 2).astype(jnp.float32) * cv.astype(jnp.float32)
                       + (iota == 3).astype(jnp.float32) * v64
                       + (iota == 4).astype(jnp.float32) * m_v[0])
        pltpu.sync_copy(orow_v, out.at[r])

    pltpu.async_copy(bmax.at[base], bmax_va, bsa)
    pltpu.async_copy(bmax.at[base + 1], bmax_vb, bsb)

    def pair_body(r2, carry):
        t_prev, m_prev = carry
        t_a, m_a = a_phase(2 * r2, bmax_va, idx_va, bsa, gsa, cand_va)

        @pl.when(r2 >= 1)
        def _():
            b_phase(2 * r2 - 1, cand_vb, gsb, t_prev, m_prev)

        t_b, m_b = a_phase(2 * r2 + 1, bmax_vb, idx_vb, bsb, gsb, cand_vb)
        b_phase(2 * r2, cand_va, gsa, t_a, m_a)
        return t_b, m_b

    t_fin, m_fin = lax.fori_loop(0, rows_per_w // 2, pair_body, (zf, zf))
    b_phase(rows_per_w - 1, cand_vb, gsb, t_fin, m_fin)


def _finish_kernel(st_ref, out_ref, *, nq):
    x = st_ref[...]
    z_gt, t_gt = x[:, 0:1], x[:, 1:2]
    c_gt, v64, m = x[:, 2:3], x[:, 3:4], x[:, 4:5]
    n_tie = jnp.float32(K_NN) - c_gt
    ut = v64 - m
    et = jnp.exp(ut)
    z = z_gt + n_tie * et
    t = t_gt + n_tie * ut * et
    ent = jnp.log(z) - t / z
    out_ref[...] = jnp.sum(ent).reshape(1, 1) * (1.0 / nq)


def kernel(feat, gallery_features):
    nq, d = feat.shape
    ng = gallery_features.shape[0]
    ng_pad = ((ng + _CHUNK - 1) // _CHUNK) * _CHUNK
    nb_rows = ng_pad // _BUCKET
    nb_pad = ((nb_rows + _L - 1) // _L) * _L
    nchunks = ng_pad // _CHUNK
    gal = jnp.pad(gallery_features, ((0, ng_pad - ng), (0, 0)))
    gal4 = gal.reshape(nb_rows, 2, _PK, d)

    pk, bmax3 = pl.pallas_call(
        functools.partial(_sims_kernel, n_real=ng),
        grid=(nchunks,),
        in_specs=[
            pl.BlockSpec((nq, d), lambda j: (0, 0)),
            pl.BlockSpec((_BPC, 1, _PK, d), lambda j: (j, 0, 0, 0)),
            pl.BlockSpec((_BPC, 1, _PK, d), lambda j: (j, 1, 0, 0)),
        ],
        out_specs=[
            pl.BlockSpec((nq, _BPC, _PK), lambda j: (0, j, 0)),
            pl.BlockSpec((1, nq, _BPC), lambda j: (j, 0, 0)),
        ],
        out_shape=[
            jax.ShapeDtypeStruct((nq, nb_rows, _PK), jnp.int32),
            jax.ShapeDtypeStruct((nchunks, nq, _BPC), jnp.float32),
        ],
    )(feat, gal4, gal4)

    bmax = jnp.pad(bmax3.transpose(1, 0, 2).reshape(nq, nb_rows),
                   ((0, 0), (0, nb_pad - nb_rows)), constant_values=_NEG)
    sims2d = pk.reshape(nq * nb_rows, _PK)

    sc_fn = functools.partial(
        pl.kernel,
        mesh=plsc.VectorSubcoreMesh(core_axis_name="c", subcore_axis_name="s"),
        compiler_params=pltpu.CompilerParams(needs_layout_passes=False),
        out_type=jax.ShapeDtypeStruct((nq, _L), jnp.float32),
        scratch_types=[
            pltpu.VMEM((nb_pad,), jnp.float32),
            pltpu.VMEM((nb_pad,), jnp.float32),
            pltpu.VMEM((K_NN + 2 * _L,), jnp.int32),
            pltpu.VMEM((K_NN,), jnp.int32),
            pltpu.VMEM((K_NN,), jnp.int32),
            pltpu.VMEM((K_NN, _PK), jnp.int32),
            pltpu.VMEM((K_NN, _PK), jnp.int32),
            pltpu.VMEM((8, K_NN * _PK // 4 + _L), jnp.float32),
            pltpu.VMEM((_L,), jnp.float32),
            pltpu.SemaphoreType.DMA,
            pltpu.SemaphoreType.DMA,
            pltpu.SemaphoreType.DMA,
            pltpu.SemaphoreType.DMA,
            pltpu.SemaphoreType.DMA,
        ],
    )(functools.partial(_sc_select, nq=nq, nb_rows=nb_rows, nb_pad=nb_pad))
    stats = sc_fn(sims2d, bmax)

    out = pl.pallas_call(
        functools.partial(_finish_kernel, nq=float(nq)),
        in_specs=[pl.BlockSpec((nq, _L), lambda: (0, 0))],
        out_specs=pl.BlockSpec((1, 1), lambda: (0, 0)),
        out_shape=jax.ShapeDtypeStruct((1, 1), jnp.float32),
    )(stats)
    return out[0, 0]


# speedup vs baseline: 25.3902x; 1.2050x over previous
import functools

import jax
import jax.numpy as jnp
from jax import lax
from jax.experimental import pallas as pl
from jax.experimental.pallas import tpu as pltpu
from jax.experimental.pallas import tpu_sc as plsc

K_NN = 64
_CHUNK = 2048
_BUCKET = 256
_PK = _BUCKET // 2
_BPC = _CHUNK // _BUCKET
_NEG = float("-inf")
_NC, _NS, _L = 2, 16, 16
_NW = _NC * _NS


def _f2key(x):
    b = lax.bitcast_convert_type(x, jnp.int32)
    mask = lax.shift_right_arithmetic(b, 31) & jnp.int32(0x7FFFFFFF)
    s = lax.bitwise_xor(b, mask)
    return lax.bitcast_convert_type(s, jnp.uint32) + jnp.uint32(0x80000000)


def _key2f(u):
    s = lax.bitcast_convert_type(u + jnp.uint32(0x80000000), jnp.int32)
    mask = lax.shift_right_arithmetic(s, 31) & jnp.int32(0x7FFFFFFF)
    b = lax.bitwise_xor(s, mask)
    return lax.bitcast_convert_type(b, jnp.float32)


def _sims_kernel(feat_ref, glo_ref, ghi_ref, pk_ref, bmax_ref, *, n_real):
    j = pl.program_id(0)
    q = feat_ref[...]
    qn = (q * lax.rsqrt(jnp.maximum(jnp.sum(q * q, axis=1, keepdims=True),
                                    1e-30))).astype(jnp.bfloat16)
    nq = q.shape[0]

    def norm16(g_ref):
        g = g_ref[...].reshape(_CHUNK // 2, q.shape[1])
        return (g * lax.rsqrt(jnp.maximum(
            jnp.sum(g * g, axis=1, keepdims=True), 1e-30))).astype(jnp.bfloat16)

    gn_lo = norm16(glo_ref)
    gn_hi = norm16(ghi_ref)
    li = lax.broadcasted_iota(jnp.int32, (nq, _PK), 1)
    bms = []
    for b in range(_BPC):
        def half(gn, col0):
            s = lax.dot_general(qn, gn[b * _PK:(b + 1) * _PK, :],
                                (((1,), (1,)), ((), ())),
                                preferred_element_type=jnp.float32)
            s = jnp.where(li + col0 < n_real, s, _NEG)
            s16 = s.astype(jnp.bfloat16)
            sf = s16.astype(jnp.float32)
            bm = jnp.max(sf, axis=1, keepdims=True)
            u = lax.bitcast_convert_type(s16, jnp.uint16).astype(jnp.uint32)
            return u, bm

        c0 = j * _CHUNK + b * _BUCKET
        u_lo, bm_lo = half(gn_lo, c0)
        u_hi, bm_hi = half(gn_hi, c0 + _PK)
        pk = lax.shift_left(u_hi, jnp.uint32(16)) | u_lo
        pk_ref[:, b, :] = lax.bitcast_convert_type(pk, jnp.int32)
        bms.append(jnp.maximum(bm_lo, bm_hi))
    bmax_ref[...] = jnp.concatenate(bms, axis=1).reshape(1, nq, _BPC)


def _sc_select(sims2d, bmax, out, bmax_va, bmax_vb, ids_v, idx_va, idx_vb,
               cand_va, cand_vb, surv_v, orow_v, bsa, bsb, gsa, gsb, sem,
               *, nq, nb_rows, nb_pad):
    wid = lax.axis_index("s") * _NC + lax.axis_index("c")
    rows_per_w = nq // _NW
    base = wid * rows_per_w
    nbv = nb_pad // _L
    iota = lax.broadcasted_iota(jnp.int32, (_L,), 0)
    ninf = jnp.full((_L,), _NEG, jnp.float32)
    zf = jnp.zeros((_L,), jnp.float32)

    def popcnt(msk):
        return plsc.all_reduce_population_count(msk)

    def to_scalar_i(vec):
        return vec[0]

    def cnt_ge_static(ref, nvreg, t):
        cs = [jnp.zeros((_L,), jnp.int32) for _ in range(4)]
        for k in range(nvreg):
            cs[k % 4] = cs[k % 4] + popcnt(ref[pl.ds(k * _L, _L)] >= t)
        return (cs[0] + cs[1]) + (cs[2] + cs[3])

    def bis_while(cnt_fn, lo0, hi0):
        def cond(carry):
            lo, hi, it = carry
            return jnp.logical_and(it < 32, (hi - lo)[0] > 1)
        def body(carry):
            lo, hi, it = carry
            mid = lo + lax.shift_right_logical(hi - lo, jnp.uint32(1))
            ge = cnt_fn(_key2f(mid)) >= K_NN
            return jnp.where(ge, mid, lo), jnp.where(ge, hi, mid), it + 1
        lo, _, _ = lax.while_loop(cond, body, (lo0, hi0, jnp.int32(0)))
        return _key2f(lo)

    def a_phase(i, bmax_v, idx_v, bsem, gsem, cand_v):
        r = base + i
        pltpu.make_async_copy(bmax.at[r], bmax_v, bsem).wait()

        mxs = [ninf, ninf]
        mns = [jnp.full((_L,), jnp.inf, jnp.float32) for _ in range(2)]
        for k in range(nbv - 1):
            v = bmax_v[pl.ds(k * _L, _L)]
            mxs[k % 2] = jnp.maximum(mxs[k % 2], v)
            mns[k % 2] = jnp.minimum(mns[k % 2], v)
        mvec = jnp.maximum(jnp.maximum(mxs[0], mxs[1]),
                           bmax_v[pl.ds((nbv - 1) * _L, _L)])
        nvec = jnp.minimum(mns[0], mns[1])
        m = mvec[0]
        lo_f = nvec[0]
        for k in range(1, _L):
            m = jnp.maximum(m, mvec[k])
            lo_f = jnp.minimum(lo_f, nvec[k])
        m_v = jnp.full((_L,), 0.0, jnp.float32) + m
        lo_v = jnp.full((_L,), 0.0, jnp.float32) + lo_f

        t_low = bis_while(lambda t: cnt_ge_static(bmax_v, nbv, t),
                          _f2key(lo_v), _f2key(m_v) + jnp.uint32(1))

        def gt_body(k, off):
            v = bmax_v[pl.ds(k * _L, _L)]
            msk = v > t_low
            plsc.store_compressed(ids_v.at[pl.ds(off, _L)], k * _L + iota,
                                  mask=msk)
            return off + to_scalar_i(popcnt(msk))

        off = lax.fori_loop(0, nbv, gt_body, jnp.int32(0), unroll=4)

        def eq_body(k, off):
            v = bmax_v[pl.ds(k * _L, _L)]
            msk = v == t_low
            @pl.when(off < K_NN)
            def _():
                plsc.store_compressed(ids_v.at[pl.ds(off, _L)],
                                      k * _L + iota, mask=msk)
            return off + to_scalar_i(popcnt(msk))

        lax.fori_loop(0, nbv, eq_body, off, unroll=4)

        for k in range(K_NN // _L):
            idx_v[pl.ds(k * _L, _L)] = ids_v[pl.ds(k * _L, _L)] + r * nb_rows
        pltpu.async_copy(sims2d.at[idx_v], cand_v, gsem)

        @pl.when(i + 2 < rows_per_w)
        def _():
            pltpu.async_copy(bmax.at[r + 2], bmax_v, bsem)
        return t_low, m_v

    def b_phase(i, cand_v, gsem, t_low, m_v):
        r = base + i
        pltpu.make_async_copy(sims2d.at[pl.ds(0, K_NN)], cand_v, gsem).wait()

        himask = jnp.full((_L,), -65536, jnp.int32)
        def f_body(k, offs):
            j2 = k >> 3
            lb = (k & 7) * _L
            new = list(offs)
            for g in range(4):
                w = cand_v[g * (K_NN // 4) + j2, pl.ds(lb, _L)]
                f_lo = lax.bitcast_convert_type(
                    lax.shift_left(w, jnp.int32(16)), jnp.float32)
                f_hi = lax.bitcast_convert_type(w & himask, jnp.float32)
                for h, fv in ((0, f_lo), (1, f_hi)):
                    sg = 2 * g + h
                    msk = fv >= t_low
                    plsc.store_compressed(surv_v.at[sg, pl.ds(new[sg], _L)],
                                          fv, mask=msk)
                    new[sg] = new[sg] + to_scalar_i(popcnt(msk))
            return tuple(new)

        z8 = (jnp.int32(0),) * 8
        offs = lax.fori_loop(0, (K_NN // 4) * (_PK // _L), f_body, z8,
                             unroll=4)
        for g in range(8):
            surv_v[g, pl.ds(offs[g], _L)] = ninf
        nvs = [lax.div(offs[g] + (_L - 1), jnp.int32(_L)) for g in range(8)]

        def cnt8(t):
            c = jnp.zeros((_L,), jnp.int32)
            for g in range(8):
                def b(k, c2):
                    return c2 + popcnt(surv_v[g, pl.ds(k * _L, _L)] >= t)
                c = lax.fori_loop(0, nvs[g], b, c)
            return c

        v64 = bis_while(cnt8, _f2key(t_low), _f2key(m_v) + jnp.uint32(1))

        def s_body_g(g):
            def s_body(k, carry):
                zv, tv, cv = carry
                v = surv_v[g, pl.ds(k * _L, _L)]
                gt = v > v64
                e = jnp.exp(v - m_v)
                zv = zv + jnp.where(gt, e, 0.0)
                tv = tv + jnp.where(gt, (v - m_v) * e, 0.0)
                cv = cv + popcnt(gt)
                return zv, tv, cv
            return s_body

        carry = (zf, zf, jnp.zeros((_L,), jnp.int32))
        for g in range(8):
            carry = lax.fori_loop(0, nvs[g], s_body_g(g), carry)
        zv, tv, cv = carry
        z_gt = zv[0]
        t_gt = tv[0]
        for k in range(1, _L):
            z_gt = z_gt + zv[k]
            t_gt = t_gt + tv[k]

        orow_v[...] = ((iota == 0).astype(jnp.float32) * z_gt
                       + (iota == 1).astype(jnp.float32) * t_gt
                       + (iota == 2).astype(jnp.float32) * cv.astype(jnp.float32)
                       + (iota == 3).astype(jnp.float32) * v64
                       + (iota == 4).astype(jnp.float32) * m_v[0])
        pltpu.sync_copy(orow_v, out.at[r])

    pltpu.async_copy(bmax.at[base], bmax_va, bsa)
    pltpu.async_copy(bmax.at[base + 1], bmax_vb, bsb)

    def pair_body(r2, carry):
        t_prev, m_prev = carry
        t_a, m_a = a_phase(2 * r2, bmax_va, idx_va, bsa, gsa, cand_va)

        @pl.when(r2 >= 1)
        def _():
            b_phase(2 * r2 - 1, cand_vb, gsb, t_prev, m_prev)

        t_b, m_b = a_phase(2 * r2 + 1, bmax_vb, idx_vb, bsb, gsb, cand_vb)
        b_phase(2 * r2, cand_va, gsa, t_a, m_a)
        return t_b, m_b

    t_fin, m_fin = lax.fori_loop(0, rows_per_w // 2, pair_body, (zf, zf))
    b_phase(rows_per_w - 1, cand_vb, gsb, t_fin, m_fin)


def _finish_kernel(st_ref, out_ref, *, nq):
    x = st_ref[...]
    z_gt, t_gt = x[:, 0:1], x[:, 1:2]
    c_gt, v64, m = x[:, 2:3], x[:, 3:4], x[:, 4:5]
    n_tie = jnp.float32(K_NN) - c_gt
    ut = v64 - m
    et = jnp.exp(ut)
    z = z_gt + n_tie * et
    t = t_gt + n_tie * ut * et
    ent = jnp.log(z) - t / z
    out_ref[...] = jnp.sum(ent).reshape(1, 1) * (1.0 / nq)


def kernel(feat, gallery_features):
    nq, d = feat.shape
    ng = gallery_features.shape[0]
    ng_pad = ((ng + _CHUNK - 1) // _CHUNK) * _CHUNK
    nb_rows = ng_pad // _BUCKET
    nb_pad = ((nb_rows + _L - 1) // _L) * _L
    nchunks = ng_pad // _CHUNK
    gal = jnp.pad(gallery_features, ((0, ng_pad - ng), (0, 0)))
    gal4 = gal.reshape(nb_rows, 2, _PK, d)

    def run_half(feat_h):
        nqh = feat_h.shape[0]
        pk, bmax3 = pl.pallas_call(
            functools.partial(_sims_kernel, n_real=ng),
            grid=(nchunks,),
            in_specs=[
                pl.BlockSpec((nqh, d), lambda j: (0, 0)),
                pl.BlockSpec((_BPC, 1, _PK, d), lambda j: (j, 0, 0, 0)),
                pl.BlockSpec((_BPC, 1, _PK, d), lambda j: (j, 1, 0, 0)),
            ],
            out_specs=[
                pl.BlockSpec((nqh, _BPC, _PK), lambda j: (0, j, 0)),
                pl.BlockSpec((1, nqh, _BPC), lambda j: (j, 0, 0)),
            ],
            out_shape=[
                jax.ShapeDtypeStruct((nqh, nb_rows, _PK), jnp.int32),
                jax.ShapeDtypeStruct((nchunks, nqh, _BPC), jnp.float32),
            ],
        )(feat_h, gal4, gal4)

        bmax = jnp.pad(bmax3.transpose(1, 0, 2).reshape(nqh, nb_rows),
                       ((0, 0), (0, nb_pad - nb_rows)), constant_values=_NEG)
        sims2d = pk.reshape(nqh * nb_rows, _PK)

        sc_fn = functools.partial(
            pl.kernel,
            mesh=plsc.VectorSubcoreMesh(core_axis_name="c",
                                        subcore_axis_name="s"),
            compiler_params=pltpu.CompilerParams(needs_layout_passes=False),
            out_type=jax.ShapeDtypeStruct((nqh, _L), jnp.float32),
            scratch_types=[
                pltpu.VMEM((nb_pad,), jnp.float32),
                pltpu.VMEM((nb_pad,), jnp.float32),
                pltpu.VMEM((K_NN + 2 * _L,), jnp.int32),
                pltpu.VMEM((K_NN,), jnp.int32),
                pltpu.VMEM((K_NN,), jnp.int32),
                pltpu.VMEM((K_NN, _PK), jnp.int32),
                pltpu.VMEM((K_NN, _PK), jnp.int32),
                pltpu.VMEM((8, K_NN * _PK // 4 + _L), jnp.float32),
                pltpu.VMEM((_L,), jnp.float32),
                pltpu.SemaphoreType.DMA,
                pltpu.SemaphoreType.DMA,
                pltpu.SemaphoreType.DMA,
                pltpu.SemaphoreType.DMA,
                pltpu.SemaphoreType.DMA,
            ],
        )(functools.partial(_sc_select, nq=nqh, nb_rows=nb_rows,
                            nb_pad=nb_pad))
        return sc_fn(sims2d, bmax)

    h = nq // 2
    stats = jnp.concatenate([run_half(feat[:h]), run_half(feat[h:])], axis=0)

    out = pl.pallas_call(
        functools.partial(_finish_kernel, nq=float(nq)),
        in_specs=[pl.BlockSpec((nq, _L), lambda: (0, 0))],
        out_specs=pl.BlockSpec((1, 1), lambda: (0, 0)),
        out_shape=jax.ShapeDtypeStruct((1, 1), jnp.float32),
    )(stats)
    return out[0, 0]


# speedup vs baseline: 27.5009x; 1.0831x over previous
import functools

import jax
import jax.numpy as jnp
from jax import lax
from jax.experimental import pallas as pl
from jax.experimental.pallas import tpu as pltpu
from jax.experimental.pallas import tpu_sc as plsc

K_NN = 64
_CHUNK = 2048
_BUCKET = 256
_PK = _BUCKET // 2
_BPC = _CHUNK // _BUCKET
_NEG = float("-inf")
_NC, _NS, _L = 2, 16, 16
_NW = _NC * _NS


def _f2key(x):
    b = lax.bitcast_convert_type(x, jnp.int32)
    mask = lax.shift_right_arithmetic(b, 31) & jnp.int32(0x7FFFFFFF)
    s = lax.bitwise_xor(b, mask)
    return lax.bitcast_convert_type(s, jnp.uint32) + jnp.uint32(0x80000000)


def _key2f(u):
    s = lax.bitcast_convert_type(u + jnp.uint32(0x80000000), jnp.int32)
    mask = lax.shift_right_arithmetic(s, 31) & jnp.int32(0x7FFFFFFF)
    b = lax.bitwise_xor(s, mask)
    return lax.bitcast_convert_type(b, jnp.float32)


def _sims_kernel(feat_ref, glo_ref, ghi_ref, pk_ref, bmax_ref, *, n_real):
    j = pl.program_id(0)
    q = feat_ref[...]
    qn = (q * lax.rsqrt(jnp.maximum(jnp.sum(q * q, axis=1, keepdims=True),
                                    1e-30))).astype(jnp.bfloat16)
    nq = q.shape[0]

    def norm16(g_ref):
        g = g_ref[...].reshape(_CHUNK // 2, q.shape[1])
        return (g * lax.rsqrt(jnp.maximum(
            jnp.sum(g * g, axis=1, keepdims=True), 1e-30))).astype(jnp.bfloat16)

    gn_lo = norm16(glo_ref)
    gn_hi = norm16(ghi_ref)
    li = lax.broadcasted_iota(jnp.int32, (nq, _PK), 1)
    bms = []
    for b in range(_BPC):
        def half(gn, col0):
            s = lax.dot_general(qn, gn[b * _PK:(b + 1) * _PK, :],
                                (((1,), (1,)), ((), ())),
                                preferred_element_type=jnp.float32)
            s = jnp.where(li + col0 < n_real, s, _NEG)
            s16 = s.astype(jnp.bfloat16)
            sf = s16.astype(jnp.float32)
            bm = jnp.max(sf, axis=1, keepdims=True)
            u = lax.bitcast_convert_type(s16, jnp.uint16).astype(jnp.uint32)
            return u, bm

        c0 = j * _CHUNK + b * _BUCKET
        u_lo, bm_lo = half(gn_lo, c0)
        u_hi, bm_hi = half(gn_hi, c0 + _PK)
        pk = lax.shift_left(u_hi, jnp.uint32(16)) | u_lo
        pk_ref[:, b, :] = lax.bitcast_convert_type(pk, jnp.int32)
        bms.append(jnp.maximum(bm_lo, bm_hi))
    bmax_ref[...] = jnp.concatenate(bms, axis=1).reshape(1, nq, _BPC)


def _sc_select(sims2d, bmax, out, bmax_va, bmax_vb, ids_v, idx_va, idx_vb,
               cand_va, cand_vb, surv_v, orow_v, bsa, bsb, gsa, gsb, sem,
               *, nq, nb_rows, nb_pad):
    wid = lax.axis_index("s") * _NC + lax.axis_index("c")
    rows_per_w = nq // _NW
    base = wid * rows_per_w
    nbv = nb_pad // _L
    iota = lax.broadcasted_iota(jnp.int32, (_L,), 0)
    ninf = jnp.full((_L,), _NEG, jnp.float32)
    zf = jnp.zeros((_L,), jnp.float32)

    def popcnt(msk):
        return plsc.all_reduce_population_count(msk)

    def to_scalar_i(vec):
        return vec[0]

    def cnt_ge_static(ref, nvreg, t):
        cs = [jnp.zeros((_L,), jnp.int32) for _ in range(4)]
        for k in range(nvreg):
            cs[k % 4] = cs[k % 4] + popcnt(ref[pl.ds(k * _L, _L)] >= t)
        return (cs[0] + cs[1]) + (cs[2] + cs[3])

    def bis_while(cnt_fn, lo0, hi0):
        def cond(carry):
            lo, hi, it = carry
            return jnp.logical_and(it < 32, (hi - lo)[0] > 1)
        def body(carry):
            lo, hi, it = carry
            mid = lo + lax.shift_right_logical(hi - lo, jnp.uint32(1))
            ge = cnt_fn(_key2f(mid)) >= K_NN
            return jnp.where(ge, mid, lo), jnp.where(ge, hi, mid), it + 1
        lo, _, _ = lax.while_loop(cond, body, (lo0, hi0, jnp.int32(0)))
        return _key2f(lo)

    def a_phase(i, bmax_v, idx_v, bsem, gsem, cand_v):
        r = base + i
        pltpu.make_async_copy(bmax.at[r], bmax_v, bsem).wait()

        mxs = [ninf, ninf]
        mns = [jnp.full((_L,), jnp.inf, jnp.float32) for _ in range(2)]
        for k in range(nbv - 1):
            v = bmax_v[pl.ds(k * _L, _L)]
            mxs[k % 2] = jnp.maximum(mxs[k % 2], v)
            mns[k % 2] = jnp.minimum(mns[k % 2], v)
        mvec = jnp.maximum(jnp.maximum(mxs[0], mxs[1]),
                           bmax_v[pl.ds((nbv - 1) * _L, _L)])
        nvec = jnp.minimum(mns[0], mns[1])
        m = mvec[0]
        lo_f = nvec[0]
        for k in range(1, _L):
            m = jnp.maximum(m, mvec[k])
            lo_f = jnp.minimum(lo_f, nvec[k])
        m_v = jnp.full((_L,), 0.0, jnp.float32) + m
        lo_v = jnp.full((_L,), 0.0, jnp.float32) + lo_f

        t_low = bis_while(lambda t: cnt_ge_static(bmax_v, nbv, t),
                          _f2key(lo_v), _f2key(m_v) + jnp.uint32(1))

        def gt_body(k, off):
            v = bmax_v[pl.ds(k * _L, _L)]
            msk = v > t_low
            plsc.store_compressed(ids_v.at[pl.ds(off, _L)], k * _L + iota,
                                  mask=msk)
            return off + to_scalar_i(popcnt(msk))

        off = lax.fori_loop(0, nbv, gt_body, jnp.int32(0), unroll=4)

        def eq_body(k, off):
            v = bmax_v[pl.ds(k * _L, _L)]
            msk = v == t_low
            @pl.when(off < K_NN)
            def _():
                plsc.store_compressed(ids_v.at[pl.ds(off, _L)],
                                      k * _L + iota, mask=msk)
            return off + to_scalar_i(popcnt(msk))

        lax.fori_loop(0, nbv, eq_body, off, unroll=4)

        for k in range(K_NN // _L):
            idx_v[pl.ds(k * _L, _L)] = ids_v[pl.ds(k * _L, _L)] + r * nb_rows
        pltpu.async_copy(sims2d.at[idx_v], cand_v, gsem)

        @pl.when(i + 2 < rows_per_w)
        def _():
            pltpu.async_copy(bmax.at[r + 2], bmax_v, bsem)
        return t_low, m_v

    def b_phase(i, cand_v, gsem, t_low, m_v):
        r = base + i
        pltpu.make_async_copy(sims2d.at[pl.ds(0, K_NN)], cand_v, gsem).wait()

        himask = jnp.full((_L,), -65536, jnp.int32)
        def f_body(k, offs):
            j2 = k >> 3
            lb = (k & 7) * _L
            new = list(offs)
            for g in range(4):
                w = cand_v[g * (K_NN // 4) + j2, pl.ds(lb, _L)]
                f_lo = lax.bitcast_convert_type(
                    lax.shift_left(w, jnp.int32(16)), jnp.float32)
                f_hi = lax.bitcast_convert_type(w & himask, jnp.float32)
                for h, fv in ((0, f_lo), (1, f_hi)):
                    sg = 2 * g + h
                    msk = fv >= t_low
                    plsc.store_compressed(surv_v.at[sg, pl.ds(new[sg], _L)],
                                          fv, mask=msk)
                    new[sg] = new[sg] + to_scalar_i(popcnt(msk))
            return tuple(new)

        z8 = (jnp.int32(0),) * 8
        offs = lax.fori_loop(0, (K_NN // 4) * (_PK // _L), f_body, z8,
                             unroll=4)
        for g in range(8):
            surv_v[g, pl.ds(offs[g], _L)] = ninf
        nvs = [lax.div(offs[g] + (_L - 1), jnp.int32(_L)) for g in range(8)]

        def cnt8(t):
            c = jnp.zeros((_L,), jnp.int32)
            for g in range(8):
                def b(k, c2):
                    return c2 + popcnt(surv_v[g, pl.ds(k * _L, _L)] >= t)
                c = lax.fori_loop(0, nvs[g], b, c)
            return c

        v64 = bis_while(cnt8, _f2key(t_low), _f2key(m_v) + jnp.uint32(1))

        def s_body_g(g):
            def s_body(k, carry):
                zv, tv, cv = carry
                v = surv_v[g, pl.ds(k * _L, _L)]
                gt = v > v64
                e = jnp.exp(v - m_v)
                zv = zv + jnp.where(gt, e, 0.0)
                tv = tv + jnp.where(gt, (v - m_v) * e, 0.0)
                cv = cv + popcnt(gt)
                return zv, tv, cv
            return s_body

        carry = (zf, zf, jnp.zeros((_L,), jnp.int32))
        for g in range(8):
            carry = lax.fori_loop(0, nvs[g], s_body_g(g), carry)
        zv, tv, cv = carry
        z_gt = zv[0]
        t_gt = tv[0]
        for k in range(1, _L):
            z_gt = z_gt + zv[k]
            t_gt = t_gt + tv[k]

        orow_v[...] = ((iota == 0).astype(jnp.float32) * z_gt
                       + (iota == 1).astype(jnp.float32) * t_gt
                       + (iota == 2).astype(jnp.float32) * cv.astype(jnp.float32)
                       + (iota == 3).astype(jnp.float32) * v64
                       + (iota == 4).astype(jnp.float32) * m_v[0])
        pltpu.sync_copy(orow_v, out.at[r])

    pltpu.async_copy(bmax.at[base], bmax_va, bsa)
    pltpu.async_copy(bmax.at[base + 1], bmax_vb, bsb)

    def pair_body(r2, carry):
        t_prev, m_prev = carry
        t_a, m_a = a_phase(2 * r2, bmax_va, idx_va, bsa, gsa, cand_va)

        @pl.when(r2 >= 1)
        def _():
            b_phase(2 * r2 - 1, cand_vb, gsb, t_prev, m_prev)

        t_b, m_b = a_phase(2 * r2 + 1, bmax_vb, idx_vb, bsb, gsb, cand_vb)
        b_phase(2 * r2, cand_va, gsa, t_a, m_a)
        return t_b, m_b

    t_fin, m_fin = lax.fori_loop(0, rows_per_w // 2, pair_body, (zf, zf))
    b_phase(rows_per_w - 1, cand_vb, gsb, t_fin, m_fin)


def _finish_kernel(st_ref, out_ref, *, nq):
    x = st_ref[...]
    z_gt, t_gt = x[:, 0:1], x[:, 1:2]
    c_gt, v64, m = x[:, 2:3], x[:, 3:4], x[:, 4:5]
    n_tie = jnp.float32(K_NN) - c_gt
    ut = v64 - m
    et = jnp.exp(ut)
    z = z_gt + n_tie * et
    t = t_gt + n_tie * ut * et
    ent = jnp.log(z) - t / z
    out_ref[...] = jnp.sum(ent).reshape(1, 1) * (1.0 / nq)


def kernel(feat, gallery_features):
    nq, d = feat.shape
    ng = gallery_features.shape[0]
    ng_pad = ((ng + _CHUNK - 1) // _CHUNK) * _CHUNK
    nb_rows = ng_pad // _BUCKET
    nb_pad = ((nb_rows + _L - 1) // _L) * _L
    nchunks = ng_pad // _CHUNK
    gal = jnp.pad(gallery_features, ((0, ng_pad - ng), (0, 0)))
    gal4 = gal.reshape(nb_rows, 2, _PK, d)

    def run_half(feat_h):
        nqh = feat_h.shape[0]
        pk, bmax3 = pl.pallas_call(
            functools.partial(_sims_kernel, n_real=ng),
            grid=(nchunks,),
            in_specs=[
                pl.BlockSpec((nqh, d), lambda j: (0, 0)),
                pl.BlockSpec((_BPC, 1, _PK, d), lambda j: (j, 0, 0, 0)),
                pl.BlockSpec((_BPC, 1, _PK, d), lambda j: (j, 1, 0, 0)),
            ],
            out_specs=[
                pl.BlockSpec((nqh, _BPC, _PK), lambda j: (0, j, 0)),
                pl.BlockSpec((1, nqh, _BPC), lambda j: (j, 0, 0)),
            ],
            out_shape=[
                jax.ShapeDtypeStruct((nqh, nb_rows, _PK), jnp.int32),
                jax.ShapeDtypeStruct((nchunks, nqh, _BPC), jnp.float32),
            ],
        )(feat_h, gal4, gal4)

        bmax = jnp.pad(bmax3.transpose(1, 0, 2).reshape(nqh, nb_rows),
                       ((0, 0), (0, nb_pad - nb_rows)), constant_values=_NEG)
        sims2d = pk.reshape(nqh * nb_rows, _PK)

        sc_fn = functools.partial(
            pl.kernel,
            mesh=plsc.VectorSubcoreMesh(core_axis_name="c",
                                        subcore_axis_name="s"),
            compiler_params=pltpu.CompilerParams(needs_layout_passes=False),
            out_type=jax.ShapeDtypeStruct((nqh, _L), jnp.float32),
            scratch_types=[
                pltpu.VMEM((nb_pad,), jnp.float32),
                pltpu.VMEM((nb_pad,), jnp.float32),
                pltpu.VMEM((K_NN + 2 * _L,), jnp.int32),
                pltpu.VMEM((K_NN,), jnp.int32),
                pltpu.VMEM((K_NN,), jnp.int32),
                pltpu.VMEM((K_NN, _PK), jnp.int32),
                pltpu.VMEM((K_NN, _PK), jnp.int32),
                pltpu.VMEM((8, K_NN * _PK // 4 + _L), jnp.float32),
                pltpu.VMEM((_L,), jnp.float32),
                pltpu.SemaphoreType.DMA,
                pltpu.SemaphoreType.DMA,
                pltpu.SemaphoreType.DMA,
                pltpu.SemaphoreType.DMA,
                pltpu.SemaphoreType.DMA,
            ],
        )(functools.partial(_sc_select, nq=nqh, nb_rows=nb_rows,
                            nb_pad=nb_pad))
        return sc_fn(sims2d, bmax)

    h = nq // 4
    stats = jnp.concatenate(
        [run_half(feat[i * h:(i + 1) * h]) for i in range(4)], axis=0)

    out = pl.pallas_call(
        functools.partial(_finish_kernel, nq=float(nq)),
        in_specs=[pl.BlockSpec((nq, _L), lambda: (0, 0))],
        out_specs=pl.BlockSpec((1, 1), lambda: (0, 0)),
        out_shape=jax.ShapeDtypeStruct((1, 1), jnp.float32),
    )(stats)
    return out[0, 0]
